# baseline (speedup 1.0000x reference)
import functools

import jax
import jax.numpy as jnp
from jax import lax
from jax.experimental import pallas as pl
from jax.experimental.pallas import tpu as pltpu
from jax.experimental.pallas import tpu_sc as plsc

F32 = jnp.float32

N = 10000
E = 320000
NS = 32
MH = 64
GR = 128

CHUNK = 128
CP = 2560
EP = CP * CHUNK
NWORK = 32
PW = CP // NWORK
NTILE = 16
NP = 12800
ACC = 10240
NR = ACC // NTILE
GB = 8
GBG = 4


def _sc_gather(h, fr2d, to2d):
    mesh = plsc.VectorSubcoreMesh(core_axis_name="c", subcore_axis_name="s")

    @functools.partial(
        pl.kernel,
        mesh=mesh,
        out_type=(jax.ShapeDtypeStruct((EP, NS), jnp.bfloat16),
                  jax.ShapeDtypeStruct((EP, NS), jnp.bfloat16)),
        scratch_types=[
            pltpu.VMEM((PW, CHUNK), jnp.int32),
            pltpu.VMEM((PW, CHUNK), jnp.int32),
            pltpu.VMEM((2 * GBG * CHUNK, NS), jnp.bfloat16),
            pltpu.VMEM((2 * GBG * CHUNK, NS), jnp.bfloat16),
            pltpu.VMEM_SHARED((N, NS), jnp.bfloat16),
            pltpu.SemaphoreType.DMA,
            pltpu.SemaphoreType.DMA,
        ],
        compiler_params=pltpu.CompilerParams(use_tc_tiling_on_sc=False),
    )
    def k(h_hbm, fr_hbm, to_hbm, of_hbm, ot_hbm, fidx, tidx, fbuf, tbuf,
          h_sh, s1, s2):
        c = lax.axis_index("c")
        s = lax.axis_index("s")
        hr = 640
        @pl.when(s < NTILE - 1)
        def _():
            pltpu.sync_copy(h_hbm.at[pl.ds(s * hr, hr)],
                            h_sh.at[pl.ds(s * hr, hr)])

        @pl.when(s == NTILE - 1)
        def _():
            rem = N - (NTILE - 1) * hr
            pltpu.sync_copy(h_hbm.at[pl.ds((NTILE - 1) * hr, rem)],
                            h_sh.at[pl.ds((NTILE - 1) * hr, rem)])

        plsc.subcore_barrier()
        wid = s * 2 + c
        base = wid * PW
        pltpu.sync_copy(fr_hbm.at[pl.ds(base, PW)], fidx)
        pltpu.sync_copy(to_hbm.at[pl.ds(base, PW)], tidx)

        half = GBG * CHUNK
        ng = PW // GBG

        def fire(g, off):
            for b in range(GBG):
                j = g * GBG + b
                pltpu.async_copy(h_sh.at[fidx.at[j]],
                                 fbuf.at[pl.ds(off + b * CHUNK, CHUNK)], s1)
                pltpu.async_copy(h_sh.at[tidx.at[j]],
                                 tbuf.at[pl.ds(off + b * CHUNK, CHUNK)], s2)

        def drain(off):
            for b in range(GBG):
                pltpu.make_async_copy(
                    of_hbm.at[pl.ds(0, CHUNK)],
                    fbuf.at[pl.ds(off + b * CHUNK, CHUNK)], s1).wait()
                pltpu.make_async_copy(
                    of_hbm.at[pl.ds(0, CHUNK)],
                    tbuf.at[pl.ds(off + b * CHUNK, CHUNK)], s2).wait()

        def write(g, off):
            pltpu.sync_copy(fbuf.at[pl.ds(off, half)],
                            of_hbm.at[pl.ds((base + g * GBG) * CHUNK, half)])
            pltpu.sync_copy(tbuf.at[pl.ds(off, half)],
                            ot_hbm.at[pl.ds((base + g * GBG) * CHUNK, half)])

        fire(0, 0)

        def pair(g2, carry):
            g = g2 * 2
            fire(g + 1, half)
            drain(0)
            write(g, 0)

            @pl.when(g + 2 < ng)
            def _():
                fire(g + 2, 0)

            drain(half)
            write(g + 1, half)
            return carry

        lax.fori_loop(0, ng // 2, pair, 0)

    return k(h, fr2d, to2d)


def _sc_scatter(m, to2d, zrows):
    mesh = plsc.VectorSubcoreMesh(core_axis_name="c", subcore_axis_name="s")

    @functools.partial(
        pl.kernel,
        mesh=mesh,
        out_type=jax.ShapeDtypeStruct((2 * NP, MH), F32),
        scratch_types=[
            pltpu.VMEM((PW, CHUNK), jnp.int32),
            pltpu.VMEM((GB * CHUNK, MH), F32),
            pltpu.VMEM_SHARED((ACC, MH), F32),
            pltpu.SemaphoreType.DMA,
        ],
        compiler_params=pltpu.CompilerParams(use_tc_tiling_on_sc=False),
    )
    def k(m_hbm, to_hbm, z_hbm, out_hbm, tidx, mbuf, acc, sem):
        c = lax.axis_index("c")
        s = lax.axis_index("s")
        pltpu.sync_copy(z_hbm.at[pl.ds(s * NR, NR)], acc.at[pl.ds(s * NR, NR)])
        plsc.subcore_barrier()
        base = (c * NTILE + s) * PW
        pltpu.sync_copy(to_hbm.at[pl.ds(base, PW)], tidx)

        def group(g, carry):
            pltpu.sync_copy(m_hbm.at[pl.ds((base + g * GB) * CHUNK, GB * CHUNK)],
                            mbuf)
            waits = []
            for b in range(GB):
                waits.append(pltpu.async_copy(
                    mbuf.at[pl.ds(b * CHUNK, CHUNK)],
                    acc.at[tidx.at[g * GB + b]], sem, add=True))
            for w in waits:
                w.wait()
            return carry

        lax.fori_loop(0, PW // GB, group, 0)
        plsc.subcore_barrier()
        pltpu.sync_copy(acc.at[pl.ds(s * NR, NR)],
                        out_hbm.at[pl.ds(c * NP + s * NR, NR)])

    return k(m, to2d, zrows)


def _encode_nodes(nf, W, b):
    n, f = nf.shape
    blk = 2000

    def body(nf_ref, w_ref, b_ref, o_ref, ob_ref):
        hv = (jnp.dot(nf_ref[...], w_ref[...],
                      preferred_element_type=F32) + b_ref[...])
        o_ref[...] = hv
        ob_ref[...] = hv.astype(jnp.bfloat16)

    return pl.pallas_call(
        body,
        grid=(n // blk,),
        in_specs=[pl.BlockSpec((blk, f), lambda i: (i, 0)),
                  pl.BlockSpec((f, NS), lambda i: (0, 0)),
                  pl.BlockSpec((1, NS), lambda i: (0, 0))],
        out_specs=[pl.BlockSpec((blk, NS), lambda i: (i, 0)),
                   pl.BlockSpec((blk, NS), lambda i: (i, 0))],
        out_shape=[jax.ShapeDtypeStruct((n, NS), F32),
                   jax.ShapeDtypeStruct((n, NS), jnp.bfloat16)],
    )(nf, W, b.reshape(1, NS))


def _message_mlp(hf, ht, ef, W1f, W1t, We, b1, W2, b2):
    blk = 5120
    grid = EP // blk
    fe = ef.shape[1]

    def body(hf_ref, ht_ref, ef_ref, w1f_ref, w1t_ref, we_ref, b1_ref,
             w2_ref, b2_ref, o_ref):
        z = (jnp.dot(hf_ref[...].astype(F32), w1f_ref[...],
                     preferred_element_type=F32)
             + jnp.dot(ht_ref[...].astype(F32), w1t_ref[...],
                       preferred_element_type=F32)
             + jnp.dot(ef_ref[...], we_ref[...], preferred_element_type=F32)
             + b1_ref[...])
        z = jnp.maximum(z, 0.0)
        m = jnp.dot(z, w2_ref[...], preferred_element_type=F32) + b2_ref[...]
        row = (lax.broadcasted_iota(jnp.int32, (blk, 1), 0)
               + pl.program_id(0) * blk)
        o_ref[...] = jnp.where(row < E, m, 0.0)

    cmap = lambda i: (0, 0)
    return pl.pallas_call(
        body,
        grid=(grid,),
        in_specs=[pl.BlockSpec((blk, NS), lambda i: (i, 0)),
                  pl.BlockSpec((blk, NS), lambda i: (i, 0)),
                  pl.BlockSpec((blk, fe), lambda i: (i, 0)),
                  pl.BlockSpec((NS, MH), cmap),
                  pl.BlockSpec((NS, MH), cmap),
                  pl.BlockSpec((fe, MH), cmap),
                  pl.BlockSpec((1, MH), cmap),
                  pl.BlockSpec((MH, MH), cmap),
                  pl.BlockSpec((1, MH), cmap)],
        out_specs=pl.BlockSpec((blk, MH), lambda i: (i, 0)),
        out_shape=jax.ShapeDtypeStruct((EP, MH), F32),
    )(hf, ht, ef, W1f, W1t, We, b1.reshape(1, MH), W2, b2.reshape(1, MH))


def _attention(h, qp, kg):
    qb = 400
    kb = 1000
    nk = N // kb
    grid = N // qb

    def body(hq_ref, hk_ref, qp_ref, kg_ref, o_ref):
        q = hq_ref[...]
        qpv = qp_ref[...]
        qmax = jnp.max(qpv)
        qmin = jnp.min(qpv)
        n0 = jnp.sum(jnp.where(kg_ref[...] == 0.0, 1, 0))
        jlo = jnp.where(qmin == 1.0, n0 // kb, 0)
        jhi = jnp.where(qmax == 0.0, (n0 + kb - 1) // kb, nk)

        def kstep(j, carry):
            kgv = kg_ref[j]
            m_i, l_i, acc = carry
            start = pl.multiple_of(j * kb, 8)
            kblk = hk_ref[pl.ds(start, kb), :]
            s = lax.dot_general(q, kblk, (((1,), (1,)), ((), ())),
                                preferred_element_type=F32)
            s = jnp.where(qpv == kgv, s, -1e9)
            m_new = jnp.maximum(m_i, jnp.max(s, axis=1, keepdims=True))
            alpha = jnp.exp(m_i - m_new)
            p = jnp.exp(s - m_new)
            l_new = l_i * alpha + jnp.sum(p, axis=1, keepdims=True)
            acc_new = acc * alpha + jnp.dot(p, kblk,
                                            preferred_element_type=F32)
            return m_new, l_new, acc_new

        m0 = jnp.full((qb, 1), -jnp.inf, F32)
        l0 = jnp.zeros((qb, 1), F32)
        a0 = jnp.zeros((qb, NS), F32)
        _, l_f, acc = lax.fori_loop(jlo, jhi, kstep, (m0, l0, a0))
        hmean = jnp.mean(hk_ref[...], axis=0, keepdims=True)
        o_ref[...] = jnp.where(l_f > 0.0, acc / jnp.maximum(l_f, 1e-30),
                               hmean)

    cmap = lambda i: (0, 0)
    return pl.pallas_call(
        body,
        grid=(grid,),
        in_specs=[pl.BlockSpec((qb, NS), lambda i: (i, 0)),
                  pl.BlockSpec((N, NS), cmap),
                  pl.BlockSpec((qb, 1), lambda i: (i, 0)),
                  pl.BlockSpec((nk, 1, kb), lambda i: (0, 0, 0))],
        out_specs=pl.BlockSpec((qb, NS), lambda i: (i, 0)),
        out_shape=jax.ShapeDtypeStruct((N, NS), F32),
    )(h, h, qp, kg.reshape(nk, 1, kb))


def _gru_update(h, att, parts, War, Waz, Wan, Wxr, Wxz, Wxn,
                Whr, Whz, Whn, br, bz, bin_, bhn):
    qb = 400
    grid = N // qb

    def body(hq_ref, att_ref, agg0_ref, agg1_ref,
             war_ref, waz_ref, wan_ref, wxr_ref, wxz_ref, wxn_ref,
             whr_ref, whz_ref, whn_ref, br_ref, bz_ref, bin_ref, bhn_ref,
             o_ref, ob_ref):
        q = hq_ref[...]
        ag = agg0_ref[...] + agg1_ref[...]
        ai = q - att_ref[...]
        pre_r = (jnp.dot(ag, war_ref[...], preferred_element_type=F32)
                 + jnp.dot(ai, wxr_ref[...], preferred_element_type=F32)
                 + jnp.dot(q, whr_ref[...], preferred_element_type=F32)
                 + br_ref[...])
        pre_z = (jnp.dot(ag, waz_ref[...], preferred_element_type=F32)
                 + jnp.dot(ai, wxz_ref[...], preferred_element_type=F32)
                 + jnp.dot(q, whz_ref[...], preferred_element_type=F32)
                 + bz_ref[...])
        i_n = (jnp.dot(ag, wan_ref[...], preferred_element_type=F32)
               + jnp.dot(ai, wxn_ref[...], preferred_element_type=F32)
               + bin_ref[...])
        h_n = jnp.dot(q, whn_ref[...], preferred_element_type=F32) + bhn_ref[...]
        r = jax.nn.sigmoid(pre_r)
        zg = jax.nn.sigmoid(pre_z)
        nn = jnp.tanh(i_n + r * h_n)
        hv = (1.0 - zg) * nn + zg * q
        o_ref[...] = hv
        ob_ref[...] = hv.astype(jnp.bfloat16)

    cmap = lambda i: (0, 0)
    np_off = NP // qb
    return pl.pallas_call(
        body,
        grid=(grid,),
        in_specs=[pl.BlockSpec((qb, NS), lambda i: (i, 0)),
                  pl.BlockSpec((qb, NS), lambda i: (i, 0)),
                  pl.BlockSpec((qb, MH), lambda i: (i, 0)),
                  pl.BlockSpec((qb, MH), lambda i: (np_off + i, 0)),
                  pl.BlockSpec((MH, NS), cmap),
                  pl.BlockSpec((MH, NS), cmap),
                  pl.BlockSpec((MH, NS), cmap),
                  pl.BlockSpec((NS, NS), cmap),
                  pl.BlockSpec((NS, NS), cmap),
                  pl.BlockSpec((NS, NS), cmap),
                  pl.BlockSpec((NS, NS), cmap),
                  pl.BlockSpec((NS, NS), cmap),
                  pl.BlockSpec((NS, NS), cmap),
                  pl.BlockSpec((1, NS), cmap),
                  pl.BlockSpec((1, NS), cmap),
                  pl.BlockSpec((1, NS), cmap),
                  pl.BlockSpec((1, NS), cmap)],
        out_specs=[pl.BlockSpec((qb, NS), lambda i: (i, 0)),
                   pl.BlockSpec((qb, NS), lambda i: (i, 0))],
        out_shape=[jax.ShapeDtypeStruct((N, NS), F32),
                   jax.ShapeDtypeStruct((N, NS), jnp.bfloat16)],
    )(h, att, parts, parts, War, Waz, Wan, Wxr, Wxz, Wxn, Whr, Whz, Whn,
      br.reshape(1, NS), bz.reshape(1, NS), bin_.reshape(1, NS),
      bhn.reshape(1, NS))


def _aggregate(h, seg, Wg_g, Wg_v, bg_g, bg_v, Wg2, bg2):
    blk = 2000
    grid = N // blk

    def body(h_ref, seg_ref, wgg_ref, wgv_ref, bgg_ref, bgv_ref,
             wg2_ref, bg2_ref, o_ref, acc_ref):
        i = pl.program_id(0)

        @pl.when(i == 0)
        def _():
            acc_ref[...] = jnp.zeros_like(acc_ref)

        hv = h_ref[...]
        g1 = jnp.dot(hv, wgg_ref[...], preferred_element_type=F32) + bgg_ref[...]
        g2 = jnp.dot(hv, wgv_ref[...], preferred_element_type=F32) + bgv_ref[...]
        gated = jax.nn.sigmoid(g1) * g2
        sv = seg_ref[...]
        w0 = jnp.where(sv == 0.0, 1.0, 0.0)
        w1 = jnp.where(sv == 1.0, 1.0, 0.0)
        s0 = jnp.sum(gated * w0, axis=0, keepdims=True)
        s1 = jnp.sum(gated * w1, axis=0, keepdims=True)
        acc_ref[0:1, :] = acc_ref[0:1, :] + s0
        acc_ref[1:2, :] = acc_ref[1:2, :] + s1

        @pl.when(i == grid - 1)
        def _():
            o_ref[...] = (jnp.dot(acc_ref[0:2, :], wg2_ref[...],
                                  preferred_element_type=F32) + bg2_ref[...])

    cmap = lambda i: (0, 0)
    return pl.pallas_call(
        body,
        grid=(grid,),
        in_specs=[pl.BlockSpec((blk, NS), lambda i: (i, 0)),
                  pl.BlockSpec((blk, 1), lambda i: (i, 0)),
                  pl.BlockSpec((NS, GR), cmap),
                  pl.BlockSpec((NS, GR), cmap),
                  pl.BlockSpec((1, GR), cmap),
                  pl.BlockSpec((1, GR), cmap),
                  pl.BlockSpec((GR, GR), cmap),
                  pl.BlockSpec((1, GR), cmap)],
        out_specs=pl.BlockSpec((2, GR), cmap),
        out_shape=jax.ShapeDtypeStruct((2, GR), F32),
        scratch_shapes=[pltpu.VMEM((8, GR), F32)],
    )(h, seg, Wg_g, Wg_v, bg_g.reshape(1, GR), bg_v.reshape(1, GR),
      Wg2, bg2.reshape(1, GR))


def kernel(node_features, edge_features, from_idx, to_idx, graph_idx,
           n_graphs, W_enc_n, b_enc_n, W_enc_e, b_enc_e, W_m1, b_m1,
           W_m2, b_m2, W_ih, W_hh, b_ih, b_hh, W_g1, b_g1, W_g2, b_g2):
    assert node_features.shape == (N, 128) and from_idx.shape == (E,)

    W1f = W_m1[:NS]
    W1t = W_m1[NS:2 * NS]
    W1e = W_m1[2 * NS:]
    We = W_enc_e @ W1e
    b1 = b_enc_e @ W1e + b_m1

    War, Waz, Wan = W_ih[:MH, :NS], W_ih[:MH, NS:2 * NS], W_ih[:MH, 2 * NS:]
    Wxr, Wxz, Wxn = W_ih[MH:, :NS], W_ih[MH:, NS:2 * NS], W_ih[MH:, 2 * NS:]
    Whr, Whz, Whn = W_hh[:, :NS], W_hh[:, NS:2 * NS], W_hh[:, 2 * NS:]
    br = b_ih[:NS] + b_hh[:NS]
    bz = b_ih[NS:2 * NS] + b_hh[NS:2 * NS]
    bin_ = b_ih[2 * NS:]
    bhn = b_hh[2 * NS:]

    Wg_g, Wg_v = W_g1[:, :GR], W_g1[:, GR:]
    bg_g, bg_v = b_g1[:GR], b_g1[GR:]

    pad = EP - E
    fr2d = jnp.concatenate([from_idx, jnp.zeros((pad,), jnp.int32)]).reshape(CP, CHUNK)
    to2d = jnp.concatenate([to_idx, jnp.zeros((pad,), jnp.int32)]).reshape(CP, CHUNK)
    ef_pad = jnp.concatenate([edge_features,
                              jnp.zeros((pad, edge_features.shape[1]), F32)])
    zrows = jnp.zeros((ACC, MH), F32)

    gi = graph_idx.astype(F32)
    qp = (graph_idx ^ 1).astype(F32).reshape(N, 1)
    kg = gi.reshape(1, N)
    seg = jnp.minimum(graph_idx, n_graphs - 1).astype(F32).reshape(N, 1)

    h, h_bf = _encode_nodes(node_features, W_enc_n, b_enc_n)
    for _ in range(2):
        hf, ht = _sc_gather(h_bf, fr2d, to2d)
        att = _attention(h, qp, kg)
        m = _message_mlp(hf, ht, ef_pad, W1f, W1t, We, b1, W_m2, b_m2)
        parts = _sc_scatter(m, to2d, zrows)
        h, h_bf = _gru_update(h, att, parts, War, Waz, Wan, Wxr, Wxz, Wxn,
                              Whr, Whz, Whn, br, bz, bin_, bhn)
    return _aggregate(h, seg, Wg_g, Wg_v, bg_g, bg_v, W_g2, b_g2)


# speedup vs baseline: 1.0254x; 1.0254x over previous
import functools

import jax
import jax.numpy as jnp
from jax import lax
from jax.experimental import pallas as pl
from jax.experimental.pallas import tpu as pltpu
from jax.experimental.pallas import tpu_sc as plsc

F32 = jnp.float32

N = 10000
E = 320000
NS = 32
MH = 64
GR = 128

CHUNK = 128
CP = 2560
EP = CP * CHUNK
NWORK = 32
PW = CP // NWORK
NTILE = 16
NP = 12800
ACC = 10240
NR = ACC // NTILE
GB = 8
GBG = 4


def _sc_gather(h, fr2d, to2d):
    mesh = plsc.VectorSubcoreMesh(core_axis_name="c", subcore_axis_name="s")

    @functools.partial(
        pl.kernel,
        mesh=mesh,
        out_type=(jax.ShapeDtypeStruct((EP, NS), F32),
                  jax.ShapeDtypeStruct((EP, NS), F32)),
        scratch_types=[
            pltpu.VMEM((PW, CHUNK), jnp.int32),
            pltpu.VMEM((PW, CHUNK), jnp.int32),
            pltpu.VMEM((2 * GBG * CHUNK, NS), F32),
            pltpu.VMEM((2 * GBG * CHUNK, NS), F32),
            pltpu.VMEM_SHARED((N, NS), F32),
            pltpu.SemaphoreType.DMA,
            pltpu.SemaphoreType.DMA,
        ],
        compiler_params=pltpu.CompilerParams(use_tc_tiling_on_sc=False),
    )
    def k(h_hbm, fr_hbm, to_hbm, of_hbm, ot_hbm, fidx, tidx, fbuf, tbuf,
          h_sh, s1, s2):
        c = lax.axis_index("c")
        s = lax.axis_index("s")
        hr = 640
        @pl.when(s < NTILE - 1)
        def _():
            pltpu.sync_copy(h_hbm.at[pl.ds(s * hr, hr)],
                            h_sh.at[pl.ds(s * hr, hr)])

        @pl.when(s == NTILE - 1)
        def _():
            rem = N - (NTILE - 1) * hr
            pltpu.sync_copy(h_hbm.at[pl.ds((NTILE - 1) * hr, rem)],
                            h_sh.at[pl.ds((NTILE - 1) * hr, rem)])

        plsc.subcore_barrier()
        wid = s * 2 + c
        base = wid * PW
        pltpu.sync_copy(fr_hbm.at[pl.ds(base, PW)], fidx)
        pltpu.sync_copy(to_hbm.at[pl.ds(base, PW)], tidx)

        half = GBG * CHUNK
        ng = PW // GBG

        def fire(g, off):
            for b in range(GBG):
                j = g * GBG + b
                pltpu.async_copy(h_sh.at[fidx.at[j]],
                                 fbuf.at[pl.ds(off + b * CHUNK, CHUNK)], s1)
                pltpu.async_copy(h_sh.at[tidx.at[j]],
                                 tbuf.at[pl.ds(off + b * CHUNK, CHUNK)], s2)

        def drain(off):
            for b in range(GBG):
                pltpu.make_async_copy(
                    of_hbm.at[pl.ds(0, CHUNK)],
                    fbuf.at[pl.ds(off + b * CHUNK, CHUNK)], s1).wait()
                pltpu.make_async_copy(
                    of_hbm.at[pl.ds(0, CHUNK)],
                    tbuf.at[pl.ds(off + b * CHUNK, CHUNK)], s2).wait()

        def write(g, off):
            pltpu.sync_copy(fbuf.at[pl.ds(off, half)],
                            of_hbm.at[pl.ds((base + g * GBG) * CHUNK, half)])
            pltpu.sync_copy(tbuf.at[pl.ds(off, half)],
                            ot_hbm.at[pl.ds((base + g * GBG) * CHUNK, half)])

        fire(0, 0)

        def pair(g2, carry):
            g = g2 * 2
            fire(g + 1, half)
            drain(0)
            write(g, 0)

            @pl.when(g + 2 < ng)
            def _():
                fire(g + 2, 0)

            drain(half)
            write(g + 1, half)
            return carry

        lax.fori_loop(0, ng // 2, pair, 0)

    return k(h, fr2d, to2d)


def _sc_scatter(m, to2d, zrows):
    mesh = plsc.VectorSubcoreMesh(core_axis_name="c", subcore_axis_name="s")

    @functools.partial(
        pl.kernel,
        mesh=mesh,
        out_type=jax.ShapeDtypeStruct((2 * NP, MH), F32),
        scratch_types=[
            pltpu.VMEM((PW, CHUNK), jnp.int32),
            pltpu.VMEM((GB * CHUNK, MH), F32),
            pltpu.VMEM_SHARED((ACC, MH), F32),
            pltpu.SemaphoreType.DMA,
        ],
        compiler_params=pltpu.CompilerParams(use_tc_tiling_on_sc=False),
    )
    def k(m_hbm, to_hbm, z_hbm, out_hbm, tidx, mbuf, acc, sem):
        c = lax.axis_index("c")
        s = lax.axis_index("s")
        pltpu.sync_copy(z_hbm.at[pl.ds(s * NR, NR)], acc.at[pl.ds(s * NR, NR)])
        plsc.subcore_barrier()
        base = (c * NTILE + s) * PW
        pltpu.sync_copy(to_hbm.at[pl.ds(base, PW)], tidx)

        def group(g, carry):
            pltpu.sync_copy(m_hbm.at[pl.ds((base + g * GB) * CHUNK, GB * CHUNK)],
                            mbuf)
            waits = []
            for b in range(GB):
                waits.append(pltpu.async_copy(
                    mbuf.at[pl.ds(b * CHUNK, CHUNK)],
                    acc.at[tidx.at[g * GB + b]], sem, add=True))
            for w in waits:
                w.wait()
            return carry

        lax.fori_loop(0, PW // GB, group, 0)
        plsc.subcore_barrier()
        pltpu.sync_copy(acc.at[pl.ds(s * NR, NR)],
                        out_hbm.at[pl.ds(c * NP + s * NR, NR)])

    return k(m, to2d, zrows)


def _encode_nodes(nf, W, b):
    n, f = nf.shape
    blk = 2000

    def body(nf_ref, w_ref, b_ref, o_ref):
        o_ref[...] = (jnp.dot(nf_ref[...], w_ref[...],
                              preferred_element_type=F32) + b_ref[...])

    return pl.pallas_call(
        body,
        grid=(n // blk,),
        in_specs=[pl.BlockSpec((blk, f), lambda i: (i, 0)),
                  pl.BlockSpec((f, NS), lambda i: (0, 0)),
                  pl.BlockSpec((1, NS), lambda i: (0, 0))],
        out_specs=pl.BlockSpec((blk, NS), lambda i: (i, 0)),
        out_shape=jax.ShapeDtypeStruct((n, NS), F32),
    )(nf, W, b.reshape(1, NS))


def _message_mlp(hf, ht, ef, W1f, W1t, We, b1, W2, b2):
    blk = 5120
    grid = EP // blk
    fe = ef.shape[1]

    def body(hf_ref, ht_ref, ef_ref, w1f_ref, w1t_ref, we_ref, b1_ref,
             w2_ref, b2_ref, o_ref):
        z = (jnp.dot(hf_ref[...], w1f_ref[...], preferred_element_type=F32)
             + jnp.dot(ht_ref[...], w1t_ref[...], preferred_element_type=F32)
             + jnp.dot(ef_ref[...], we_ref[...], preferred_element_type=F32)
             + b1_ref[...])
        z = jnp.maximum(z, 0.0)
        m = jnp.dot(z, w2_ref[...], preferred_element_type=F32) + b2_ref[...]
        row = (lax.broadcasted_iota(jnp.int32, (blk, 1), 0)
               + pl.program_id(0) * blk)
        o_ref[...] = jnp.where(row < E, m, 0.0)

    cmap = lambda i: (0, 0)
    return pl.pallas_call(
        body,
        grid=(grid,),
        in_specs=[pl.BlockSpec((blk, NS), lambda i: (i, 0)),
                  pl.BlockSpec((blk, NS), lambda i: (i, 0)),
                  pl.BlockSpec((blk, fe), lambda i: (i, 0)),
                  pl.BlockSpec((NS, MH), cmap),
                  pl.BlockSpec((NS, MH), cmap),
                  pl.BlockSpec((fe, MH), cmap),
                  pl.BlockSpec((1, MH), cmap),
                  pl.BlockSpec((MH, MH), cmap),
                  pl.BlockSpec((1, MH), cmap)],
        out_specs=pl.BlockSpec((blk, MH), lambda i: (i, 0)),
        out_shape=jax.ShapeDtypeStruct((EP, MH), F32),
    )(hf, ht, ef, W1f, W1t, We, b1.reshape(1, MH), W2, b2.reshape(1, MH))


def _attention(h, qp, kg, off, rows):
    qb = 400
    kb = 1000
    nk = N // kb
    grid = rows // qb

    def body(hq_ref, hk_ref, qp_ref, kg_ref, o_ref):
        q = hq_ref[...]
        qpv = qp_ref[...]
        qmax = jnp.max(qpv)
        qmin = jnp.min(qpv)
        n0 = jnp.sum(jnp.where(kg_ref[...] == 0.0, 1, 0))
        jlo = jnp.where(qmin == 1.0, n0 // kb, 0)
        jhi = jnp.where(qmax == 0.0, (n0 + kb - 1) // kb, nk)

        def kstep(j, carry):
            kgv = kg_ref[j]
            m_i, l_i, acc = carry
            start = pl.multiple_of(j * kb, 8)
            kblk = hk_ref[pl.ds(start, kb), :]
            s = lax.dot_general(q, kblk, (((1,), (1,)), ((), ())),
                                preferred_element_type=F32)
            s = jnp.where(qpv == kgv, s, -1e9)
            m_new = jnp.maximum(m_i, jnp.max(s, axis=1, keepdims=True))
            alpha = jnp.exp(m_i - m_new)
            p = jnp.exp(s - m_new)
            l_new = l_i * alpha + jnp.sum(p, axis=1, keepdims=True)
            acc_new = acc * alpha + jnp.dot(p, kblk,
                                            preferred_element_type=F32)
            return m_new, l_new, acc_new

        m0 = jnp.full((qb, 1), -jnp.inf, F32)
        l0 = jnp.zeros((qb, 1), F32)
        a0 = jnp.zeros((qb, NS), F32)
        _, l_f, acc = lax.fori_loop(jlo, jhi, kstep, (m0, l0, a0))
        hmean = jnp.mean(hk_ref[...], axis=0, keepdims=True)
        o_ref[...] = jnp.where(l_f > 0.0, acc / jnp.maximum(l_f, 1e-30),
                               hmean)

    cmap = lambda i: (0, 0)
    return pl.pallas_call(
        body,
        grid=(grid,),
        in_specs=[pl.BlockSpec((qb, NS), lambda i: (off + i, 0)),
                  pl.BlockSpec((N, NS), cmap),
                  pl.BlockSpec((qb, 1), lambda i: (off + i, 0)),
                  pl.BlockSpec((nk, 1, kb), lambda i: (0, 0, 0))],
        out_specs=pl.BlockSpec((qb, NS), lambda i: (i, 0)),
        out_shape=jax.ShapeDtypeStruct((rows, NS), F32),
    )(h, h, qp, kg.reshape(nk, 1, kb))


def _gru_update(h, att, parts, War, Waz, Wan, Wxr, Wxz, Wxn,
                Whr, Whz, Whn, br, bz, bin_, bhn):
    qb = 400
    grid = N // qb

    def body(hq_ref, att_ref, agg0_ref, agg1_ref,
             war_ref, waz_ref, wan_ref, wxr_ref, wxz_ref, wxn_ref,
             whr_ref, whz_ref, whn_ref, br_ref, bz_ref, bin_ref, bhn_ref,
             o_ref):
        q = hq_ref[...]
        ag = agg0_ref[...] + agg1_ref[...]
        ai = q - att_ref[...]
        pre_r = (jnp.dot(ag, war_ref[...], preferred_element_type=F32)
                 + jnp.dot(ai, wxr_ref[...], preferred_element_type=F32)
                 + jnp.dot(q, whr_ref[...], preferred_element_type=F32)
                 + br_ref[...])
        pre_z = (jnp.dot(ag, waz_ref[...], preferred_element_type=F32)
                 + jnp.dot(ai, wxz_ref[...], preferred_element_type=F32)
                 + jnp.dot(q, whz_ref[...], preferred_element_type=F32)
                 + bz_ref[...])
        i_n = (jnp.dot(ag, wan_ref[...], preferred_element_type=F32)
               + jnp.dot(ai, wxn_ref[...], preferred_element_type=F32)
               + bin_ref[...])
        h_n = jnp.dot(q, whn_ref[...], preferred_element_type=F32) + bhn_ref[...]
        r = jax.nn.sigmoid(pre_r)
        zg = jax.nn.sigmoid(pre_z)
        nn = jnp.tanh(i_n + r * h_n)
        o_ref[...] = (1.0 - zg) * nn + zg * q

    cmap = lambda i: (0, 0)
    np_off = NP // qb
    return pl.pallas_call(
        body,
        grid=(grid,),
        in_specs=[pl.BlockSpec((qb, NS), lambda i: (i, 0)),
                  pl.BlockSpec((qb, NS), lambda i: (i, 0)),
                  pl.BlockSpec((qb, MH), lambda i: (i, 0)),
                  pl.BlockSpec((qb, MH), lambda i: (np_off + i, 0)),
                  pl.BlockSpec((MH, NS), cmap),
                  pl.BlockSpec((MH, NS), cmap),
                  pl.BlockSpec((MH, NS), cmap),
                  pl.BlockSpec((NS, NS), cmap),
                  pl.BlockSpec((NS, NS), cmap),
                  pl.BlockSpec((NS, NS), cmap),
                  pl.BlockSpec((NS, NS), cmap),
                  pl.BlockSpec((NS, NS), cmap),
                  pl.BlockSpec((NS, NS), cmap),
                  pl.BlockSpec((1, NS), cmap),
                  pl.BlockSpec((1, NS), cmap),
                  pl.BlockSpec((1, NS), cmap),
                  pl.BlockSpec((1, NS), cmap)],
        out_specs=pl.BlockSpec((qb, NS), lambda i: (i, 0)),
        out_shape=jax.ShapeDtypeStruct((N, NS), F32),
    )(h, att, parts, parts, War, Waz, Wan, Wxr, Wxz, Wxn, Whr, Whz, Whn,
      br.reshape(1, NS), bz.reshape(1, NS), bin_.reshape(1, NS),
      bhn.reshape(1, NS))


def _aggregate(h, seg, Wg_g, Wg_v, bg_g, bg_v, Wg2, bg2):
    blk = 2000
    grid = N // blk

    def body(h_ref, seg_ref, wgg_ref, wgv_ref, bgg_ref, bgv_ref,
             wg2_ref, bg2_ref, o_ref, acc_ref):
        i = pl.program_id(0)

        @pl.when(i == 0)
        def _():
            acc_ref[...] = jnp.zeros_like(acc_ref)

        hv = h_ref[...]
        g1 = jnp.dot(hv, wgg_ref[...], preferred_element_type=F32) + bgg_ref[...]
        g2 = jnp.dot(hv, wgv_ref[...], preferred_element_type=F32) + bgv_ref[...]
        gated = jax.nn.sigmoid(g1) * g2
        sv = seg_ref[...]
        w0 = jnp.where(sv == 0.0, 1.0, 0.0)
        w1 = jnp.where(sv == 1.0, 1.0, 0.0)
        s0 = jnp.sum(gated * w0, axis=0, keepdims=True)
        s1 = jnp.sum(gated * w1, axis=0, keepdims=True)
        acc_ref[0:1, :] = acc_ref[0:1, :] + s0
        acc_ref[1:2, :] = acc_ref[1:2, :] + s1

        @pl.when(i == grid - 1)
        def _():
            o_ref[...] = (jnp.dot(acc_ref[0:2, :], wg2_ref[...],
                                  preferred_element_type=F32) + bg2_ref[...])

    cmap = lambda i: (0, 0)
    return pl.pallas_call(
        body,
        grid=(grid,),
        in_specs=[pl.BlockSpec((blk, NS), lambda i: (i, 0)),
                  pl.BlockSpec((blk, 1), lambda i: (i, 0)),
                  pl.BlockSpec((NS, GR), cmap),
                  pl.BlockSpec((NS, GR), cmap),
                  pl.BlockSpec((1, GR), cmap),
                  pl.BlockSpec((1, GR), cmap),
                  pl.BlockSpec((GR, GR), cmap),
                  pl.BlockSpec((1, GR), cmap)],
        out_specs=pl.BlockSpec((2, GR), cmap),
        out_shape=jax.ShapeDtypeStruct((2, GR), F32),
        scratch_shapes=[pltpu.VMEM((8, GR), F32)],
    )(h, seg, Wg_g, Wg_v, bg_g.reshape(1, GR), bg_v.reshape(1, GR),
      Wg2, bg2.reshape(1, GR))


def kernel(node_features, edge_features, from_idx, to_idx, graph_idx,
           n_graphs, W_enc_n, b_enc_n, W_enc_e, b_enc_e, W_m1, b_m1,
           W_m2, b_m2, W_ih, W_hh, b_ih, b_hh, W_g1, b_g1, W_g2, b_g2):
    assert node_features.shape == (N, 128) and from_idx.shape == (E,)

    W1f = W_m1[:NS]
    W1t = W_m1[NS:2 * NS]
    W1e = W_m1[2 * NS:]
    We = W_enc_e @ W1e
    b1 = b_enc_e @ W1e + b_m1

    War, Waz, Wan = W_ih[:MH, :NS], W_ih[:MH, NS:2 * NS], W_ih[:MH, 2 * NS:]
    Wxr, Wxz, Wxn = W_ih[MH:, :NS], W_ih[MH:, NS:2 * NS], W_ih[MH:, 2 * NS:]
    Whr, Whz, Whn = W_hh[:, :NS], W_hh[:, NS:2 * NS], W_hh[:, 2 * NS:]
    br = b_ih[:NS] + b_hh[:NS]
    bz = b_ih[NS:2 * NS] + b_hh[NS:2 * NS]
    bin_ = b_ih[2 * NS:]
    bhn = b_hh[2 * NS:]

    Wg_g, Wg_v = W_g1[:, :GR], W_g1[:, GR:]
    bg_g, bg_v = b_g1[:GR], b_g1[GR:]

    pad = EP - E
    fr2d = jnp.concatenate([from_idx, jnp.zeros((pad,), jnp.int32)]).reshape(CP, CHUNK)
    to2d = jnp.concatenate([to_idx, jnp.zeros((pad,), jnp.int32)]).reshape(CP, CHUNK)
    ef_pad = jnp.concatenate([edge_features,
                              jnp.zeros((pad, edge_features.shape[1]), F32)])
    zrows = jnp.zeros((ACC, MH), F32)

    gi = graph_idx.astype(F32)
    qp = (graph_idx ^ 1).astype(F32).reshape(N, 1)
    kg = gi.reshape(1, N)
    seg = jnp.minimum(graph_idx, n_graphs - 1).astype(F32).reshape(N, 1)

    h = _encode_nodes(node_features, W_enc_n, b_enc_n)
    for _ in range(2):
        hf, ht = _sc_gather(h, fr2d, to2d)
        att_a = _attention(h, qp, kg, 0, 5200)
        att_b = _attention(h, qp, kg, 13, 4800)
        att = jnp.concatenate([att_a, att_b])
        m = _message_mlp(hf, ht, ef_pad, W1f, W1t, We, b1, W_m2, b_m2)
        parts = _sc_scatter(m, to2d, zrows)
        h = _gru_update(h, att, parts, War, Waz, Wan, Wxr, Wxz, Wxn,
                        Whr, Whz, Whn, br, bz, bin_, bhn)
    return _aggregate(h, seg, Wg_g, Wg_v, bg_g, bg_v, W_g2, b_g2)


# speedup vs baseline: 1.0346x; 1.0090x over previous
import functools

import jax
import jax.numpy as jnp
from jax import lax
from jax.experimental import pallas as pl
from jax.experimental.pallas import tpu as pltpu
from jax.experimental.pallas import tpu_sc as plsc

F32 = jnp.float32

N = 10000
E = 320000
NS = 32
MH = 64
GR = 128

CHUNK = 128
CP = 2560
EP = CP * CHUNK
NWORK = 32
PW = CP // NWORK
NTILE = 16
NP = 12800
ACC = 10240
NR = ACC // NTILE
GB = 8
GBG = 4


def _sc_gather(h, fr2d, to2d):
    mesh = plsc.VectorSubcoreMesh(core_axis_name="c", subcore_axis_name="s")

    @functools.partial(
        pl.kernel,
        mesh=mesh,
        out_type=(jax.ShapeDtypeStruct((EP, NS), F32),
                  jax.ShapeDtypeStruct((EP, NS), F32)),
        scratch_types=[
            pltpu.VMEM((PW, CHUNK), jnp.int32),
            pltpu.VMEM((PW, CHUNK), jnp.int32),
            pltpu.VMEM((2 * GBG * CHUNK, NS), F32),
            pltpu.VMEM((2 * GBG * CHUNK, NS), F32),
            pltpu.VMEM_SHARED((N, NS), F32),
            pltpu.SemaphoreType.DMA,
            pltpu.SemaphoreType.DMA,
        ],
        compiler_params=pltpu.CompilerParams(use_tc_tiling_on_sc=False),
    )
    def k(h_hbm, fr_hbm, to_hbm, of_hbm, ot_hbm, fidx, tidx, fbuf, tbuf,
          h_sh, s1, s2):
        c = lax.axis_index("c")
        s = lax.axis_index("s")
        hr = 640
        @pl.when(s < NTILE - 1)
        def _():
            pltpu.sync_copy(h_hbm.at[pl.ds(s * hr, hr)],
                            h_sh.at[pl.ds(s * hr, hr)])

        @pl.when(s == NTILE - 1)
        def _():
            rem = N - (NTILE - 1) * hr
            pltpu.sync_copy(h_hbm.at[pl.ds((NTILE - 1) * hr, rem)],
                            h_sh.at[pl.ds((NTILE - 1) * hr, rem)])

        plsc.subcore_barrier()
        wid = s * 2 + c
        base = wid * PW
        pltpu.sync_copy(fr_hbm.at[pl.ds(base, PW)], fidx)
        pltpu.sync_copy(to_hbm.at[pl.ds(base, PW)], tidx)

        half = GBG * CHUNK
        ng = PW // GBG

        def fire(g, off):
            for b in range(GBG):
                j = g * GBG + b
                pltpu.async_copy(h_sh.at[fidx.at[j]],
                                 fbuf.at[pl.ds(off + b * CHUNK, CHUNK)], s1)
                pltpu.async_copy(h_sh.at[tidx.at[j]],
                                 tbuf.at[pl.ds(off + b * CHUNK, CHUNK)], s2)

        def drain(off):
            for b in range(GBG):
                pltpu.make_async_copy(
                    of_hbm.at[pl.ds(0, CHUNK)],
                    fbuf.at[pl.ds(off + b * CHUNK, CHUNK)], s1).wait()
                pltpu.make_async_copy(
                    of_hbm.at[pl.ds(0, CHUNK)],
                    tbuf.at[pl.ds(off + b * CHUNK, CHUNK)], s2).wait()

        def write(g, off):
            pltpu.sync_copy(fbuf.at[pl.ds(off, half)],
                            of_hbm.at[pl.ds((base + g * GBG) * CHUNK, half)])
            pltpu.sync_copy(tbuf.at[pl.ds(off, half)],
                            ot_hbm.at[pl.ds((base + g * GBG) * CHUNK, half)])

        fire(0, 0)

        def pair(g2, carry):
            g = g2 * 2
            fire(g + 1, half)
            drain(0)
            write(g, 0)

            @pl.when(g + 2 < ng)
            def _():
                fire(g + 2, 0)

            drain(half)
            write(g + 1, half)
            return carry

        lax.fori_loop(0, ng // 2, pair, 0)

    return k(h, fr2d, to2d)


def _sc_scatter(m, to2d, zrows):
    mesh = plsc.VectorSubcoreMesh(core_axis_name="c", subcore_axis_name="s")

    @functools.partial(
        pl.kernel,
        mesh=mesh,
        out_type=jax.ShapeDtypeStruct((2 * NP, MH), F32),
        scratch_types=[
            pltpu.VMEM((PW, CHUNK), jnp.int32),
            pltpu.VMEM((2 * GBG * CHUNK, MH), F32),
            pltpu.VMEM_SHARED((ACC, MH), F32),
            pltpu.SemaphoreType.DMA,
            pltpu.SemaphoreType.DMA,
        ],
        compiler_params=pltpu.CompilerParams(use_tc_tiling_on_sc=False),
    )
    def k(m_hbm, to_hbm, z_hbm, out_hbm, tidx, mbuf, acc, sem, seml):
        c = lax.axis_index("c")
        s = lax.axis_index("s")
        pltpu.sync_copy(z_hbm.at[pl.ds(s * NR, NR)], acc.at[pl.ds(s * NR, NR)])
        plsc.subcore_barrier()
        base = (c * NTILE + s) * PW
        pltpu.sync_copy(to_hbm.at[pl.ds(base, PW)], tidx)

        half = GBG * CHUNK
        ngs = PW // GBG

        def load(g, off):
            pltpu.async_copy(m_hbm.at[pl.ds((base + g * GBG) * CHUNK, half)],
                             mbuf.at[pl.ds(off, half)], seml)

        def drain_load(off):
            pltpu.make_async_copy(m_hbm.at[pl.ds(0, half)],
                                  mbuf.at[pl.ds(off, half)], seml).wait()

        def adds(g, off):
            waits = []
            for b in range(GBG):
                waits.append(pltpu.async_copy(
                    mbuf.at[pl.ds(off + b * CHUNK, CHUNK)],
                    acc.at[tidx.at[g * GBG + b]], sem, add=True))
            for w in waits:
                w.wait()

        load(0, 0)

        def pair(g2, carry):
            g = g2 * 2
            load(g + 1, half)
            drain_load(0)
            adds(g, 0)

            @pl.when(g + 2 < ngs)
            def _():
                load(g + 2, 0)

            drain_load(half)
            adds(g + 1, half)
            return carry

        lax.fori_loop(0, ngs // 2, pair, 0)
        plsc.subcore_barrier()
        pltpu.sync_copy(acc.at[pl.ds(s * NR, NR)],
                        out_hbm.at[pl.ds(c * NP + s * NR, NR)])

    return k(m, to2d, zrows)


def _encode_nodes(nf, W, b):
    n, f = nf.shape
    blk = 2000

    def body(nf_ref, w_ref, b_ref, o_ref):
        o_ref[...] = (jnp.dot(nf_ref[...], w_ref[...],
                              preferred_element_type=F32) + b_ref[...])

    return pl.pallas_call(
        body,
        grid=(n // blk,),
        in_specs=[pl.BlockSpec((blk, f), lambda i: (i, 0)),
                  pl.BlockSpec((f, NS), lambda i: (0, 0)),
                  pl.BlockSpec((1, NS), lambda i: (0, 0))],
        out_specs=pl.BlockSpec((blk, NS), lambda i: (i, 0)),
        out_shape=jax.ShapeDtypeStruct((n, NS), F32),
    )(nf, W, b.reshape(1, NS))


def _message_mlp(hf, ht, ef, W1f, W1t, We, b1, W2, b2):
    blk = 5120
    grid = EP // blk
    fe = ef.shape[1]

    def body(hf_ref, ht_ref, ef_ref, w1f_ref, w1t_ref, we_ref, b1_ref,
             w2_ref, b2_ref, o_ref):
        z = (jnp.dot(hf_ref[...], w1f_ref[...], preferred_element_type=F32)
             + jnp.dot(ht_ref[...], w1t_ref[...], preferred_element_type=F32)
             + jnp.dot(ef_ref[...], we_ref[...], preferred_element_type=F32)
             + b1_ref[...])
        z = jnp.maximum(z, 0.0)
        m = jnp.dot(z, w2_ref[...], preferred_element_type=F32) + b2_ref[...]
        row = (lax.broadcasted_iota(jnp.int32, (blk, 1), 0)
               + pl.program_id(0) * blk)
        o_ref[...] = jnp.where(row < E, m, 0.0)

    cmap = lambda i: (0, 0)
    return pl.pallas_call(
        body,
        grid=(grid,),
        in_specs=[pl.BlockSpec((blk, NS), lambda i: (i, 0)),
                  pl.BlockSpec((blk, NS), lambda i: (i, 0)),
                  pl.BlockSpec((blk, fe), lambda i: (i, 0)),
                  pl.BlockSpec((NS, MH), cmap),
                  pl.BlockSpec((NS, MH), cmap),
                  pl.BlockSpec((fe, MH), cmap),
                  pl.BlockSpec((1, MH), cmap),
                  pl.BlockSpec((MH, MH), cmap),
                  pl.BlockSpec((1, MH), cmap)],
        out_specs=pl.BlockSpec((blk, MH), lambda i: (i, 0)),
        out_shape=jax.ShapeDtypeStruct((EP, MH), F32),
    )(hf, ht, ef, W1f, W1t, We, b1.reshape(1, MH), W2, b2.reshape(1, MH))


def _attention(h, qp, kg):
    qb = 400
    kb = 1000
    nk = N // kb
    grid = N // qb

    def body(hq_ref, hk_ref, qp_ref, kg_ref, o_ref):
        q = hq_ref[...]
        qpv = qp_ref[...]
        qmax = jnp.max(qpv)
        qmin = jnp.min(qpv)
        n0 = jnp.sum(jnp.where(kg_ref[...] == 0.0, 1, 0))
        jlo = jnp.where(qmin == 1.0, n0 // kb, 0)
        jhi = jnp.where(qmax == 0.0, (n0 + kb - 1) // kb, nk)

        def kstep(j, carry):
            kgv = kg_ref[j]
            m_i, l_i, acc = carry
            start = pl.multiple_of(j * kb, 8)
            kblk = hk_ref[pl.ds(start, kb), :]
            s = lax.dot_general(q, kblk, (((1,), (1,)), ((), ())),
                                preferred_element_type=F32)
            s = jnp.where(qpv == kgv, s, -1e9)
            m_new = jnp.maximum(m_i, jnp.max(s, axis=1, keepdims=True))
            alpha = jnp.exp(m_i - m_new)
            p = jnp.exp(s - m_new)
            l_new = l_i * alpha + jnp.sum(p, axis=1, keepdims=True)
            acc_new = acc * alpha + jnp.dot(p, kblk,
                                            preferred_element_type=F32)
            return m_new, l_new, acc_new

        m0 = jnp.full((qb, 1), -jnp.inf, F32)
        l0 = jnp.zeros((qb, 1), F32)
        a0 = jnp.zeros((qb, NS), F32)
        _, l_f, acc = lax.fori_loop(jlo, jhi, kstep, (m0, l0, a0))
        hmean = jnp.mean(hk_ref[...], axis=0, keepdims=True)
        o_ref[...] = jnp.where(l_f > 0.0, acc / jnp.maximum(l_f, 1e-30),
                               hmean)

    cmap = lambda i: (0, 0)
    return pl.pallas_call(
        body,
        grid=(grid,),
        in_specs=[pl.BlockSpec((qb, NS), lambda i: (i, 0)),
                  pl.BlockSpec((N, NS), cmap),
                  pl.BlockSpec((qb, 1), lambda i: (i, 0)),
                  pl.BlockSpec((nk, 1, kb), lambda i: (0, 0, 0))],
        out_specs=pl.BlockSpec((qb, NS), lambda i: (i, 0)),
        out_shape=jax.ShapeDtypeStruct((N, NS), F32),
    )(h, h, qp, kg.reshape(nk, 1, kb))


def _gru_update(h, att, parts, War, Waz, Wan, Wxr, Wxz, Wxn,
                Whr, Whz, Whn, br, bz, bin_, bhn):
    qb = 400
    grid = N // qb

    def body(hq_ref, att_ref, agg0_ref, agg1_ref,
             war_ref, waz_ref, wan_ref, wxr_ref, wxz_ref, wxn_ref,
             whr_ref, whz_ref, whn_ref, br_ref, bz_ref, bin_ref, bhn_ref,
             o_ref):
        q = hq_ref[...]
        ag = agg0_ref[...] + agg1_ref[...]
        ai = q - att_ref[...]
        pre_r = (jnp.dot(ag, war_ref[...], preferred_element_type=F32)
                 + jnp.dot(ai, wxr_ref[...], preferred_element_type=F32)
                 + jnp.dot(q, whr_ref[...], preferred_element_type=F32)
                 + br_ref[...])
        pre_z = (jnp.dot(ag, waz_ref[...], preferred_element_type=F32)
                 + jnp.dot(ai, wxz_ref[...], preferred_element_type=F32)
                 + jnp.dot(q, whz_ref[...], preferred_element_type=F32)
                 + bz_ref[...])
        i_n = (jnp.dot(ag, wan_ref[...], preferred_element_type=F32)
               + jnp.dot(ai, wxn_ref[...], preferred_element_type=F32)
               + bin_ref[...])
        h_n = jnp.dot(q, whn_ref[...], preferred_element_type=F32) + bhn_ref[...]
        r = jax.nn.sigmoid(pre_r)
        zg = jax.nn.sigmoid(pre_z)
        nn = jnp.tanh(i_n + r * h_n)
        o_ref[...] = (1.0 - zg) * nn + zg * q

    cmap = lambda i: (0, 0)
    np_off = NP // qb
    return pl.pallas_call(
        body,
        grid=(grid,),
        in_specs=[pl.BlockSpec((qb, NS), lambda i: (i, 0)),
                  pl.BlockSpec((qb, NS), lambda i: (i, 0)),
                  pl.BlockSpec((qb, MH), lambda i: (i, 0)),
                  pl.BlockSpec((qb, MH), lambda i: (np_off + i, 0)),
                  pl.BlockSpec((MH, NS), cmap),
                  pl.BlockSpec((MH, NS), cmap),
                  pl.BlockSpec((MH, NS), cmap),
                  pl.BlockSpec((NS, NS), cmap),
                  pl.BlockSpec((NS, NS), cmap),
                  pl.BlockSpec((NS, NS), cmap),
                  pl.BlockSpec((NS, NS), cmap),
                  pl.BlockSpec((NS, NS), cmap),
                  pl.BlockSpec((NS, NS), cmap),
                  pl.BlockSpec((1, NS), cmap),
                  pl.BlockSpec((1, NS), cmap),
                  pl.BlockSpec((1, NS), cmap),
                  pl.BlockSpec((1, NS), cmap)],
        out_specs=pl.BlockSpec((qb, NS), lambda i: (i, 0)),
        out_shape=jax.ShapeDtypeStruct((N, NS), F32),
    )(h, att, parts, parts, War, Waz, Wan, Wxr, Wxz, Wxn, Whr, Whz, Whn,
      br.reshape(1, NS), bz.reshape(1, NS), bin_.reshape(1, NS),
      bhn.reshape(1, NS))


def _aggregate(h, seg, Wg_g, Wg_v, bg_g, bg_v, Wg2, bg2):
    blk = 2000
    grid = N // blk

    def body(h_ref, seg_ref, wgg_ref, wgv_ref, bgg_ref, bgv_ref,
             wg2_ref, bg2_ref, o_ref, acc_ref):
        i = pl.program_id(0)

        @pl.when(i == 0)
        def _():
            acc_ref[...] = jnp.zeros_like(acc_ref)

        hv = h_ref[...]
        g1 = jnp.dot(hv, wgg_ref[...], preferred_element_type=F32) + bgg_ref[...]
        g2 = jnp.dot(hv, wgv_ref[...], preferred_element_type=F32) + bgv_ref[...]
        gated = jax.nn.sigmoid(g1) * g2
        sv = seg_ref[...]
        w0 = jnp.where(sv == 0.0, 1.0, 0.0)
        w1 = jnp.where(sv == 1.0, 1.0, 0.0)
        s0 = jnp.sum(gated * w0, axis=0, keepdims=True)
        s1 = jnp.sum(gated * w1, axis=0, keepdims=True)
        acc_ref[0:1, :] = acc_ref[0:1, :] + s0
        acc_ref[1:2, :] = acc_ref[1:2, :] + s1

        @pl.when(i == grid - 1)
        def _():
            o_ref[...] = (jnp.dot(acc_ref[0:2, :], wg2_ref[...],
                                  preferred_element_type=F32) + bg2_ref[...])

    cmap = lambda i: (0, 0)
    return pl.pallas_call(
        body,
        grid=(grid,),
        in_specs=[pl.BlockSpec((blk, NS), lambda i: (i, 0)),
                  pl.BlockSpec((blk, 1), lambda i: (i, 0)),
                  pl.BlockSpec((NS, GR), cmap),
                  pl.BlockSpec((NS, GR), cmap),
                  pl.BlockSpec((1, GR), cmap),
                  pl.BlockSpec((1, GR), cmap),
                  pl.BlockSpec((GR, GR), cmap),
                  pl.BlockSpec((1, GR), cmap)],
        out_specs=pl.BlockSpec((2, GR), cmap),
        out_shape=jax.ShapeDtypeStruct((2, GR), F32),
        scratch_shapes=[pltpu.VMEM((8, GR), F32)],
    )(h, seg, Wg_g, Wg_v, bg_g.reshape(1, GR), bg_v.reshape(1, GR),
      Wg2, bg2.reshape(1, GR))


def kernel(node_features, edge_features, from_idx, to_idx, graph_idx,
           n_graphs, W_enc_n, b_enc_n, W_enc_e, b_enc_e, W_m1, b_m1,
           W_m2, b_m2, W_ih, W_hh, b_ih, b_hh, W_g1, b_g1, W_g2, b_g2):
    assert node_features.shape == (N, 128) and from_idx.shape == (E,)

    W1f = W_m1[:NS]
    W1t = W_m1[NS:2 * NS]
    W1e = W_m1[2 * NS:]
    We = W_enc_e @ W1e
    b1 = b_enc_e @ W1e + b_m1

    War, Waz, Wan = W_ih[:MH, :NS], W_ih[:MH, NS:2 * NS], W_ih[:MH, 2 * NS:]
    Wxr, Wxz, Wxn = W_ih[MH:, :NS], W_ih[MH:, NS:2 * NS], W_ih[MH:, 2 * NS:]
    Whr, Whz, Whn = W_hh[:, :NS], W_hh[:, NS:2 * NS], W_hh[:, 2 * NS:]
    br = b_ih[:NS] + b_hh[:NS]
    bz = b_ih[NS:2 * NS] + b_hh[NS:2 * NS]
    bin_ = b_ih[2 * NS:]
    bhn = b_hh[2 * NS:]

    Wg_g, Wg_v = W_g1[:, :GR], W_g1[:, GR:]
    bg_g, bg_v = b_g1[:GR], b_g1[GR:]

    pad = EP - E
    fr2d = jnp.concatenate([from_idx, jnp.zeros((pad,), jnp.int32)]).reshape(CP, CHUNK)
    to2d = jnp.concatenate([to_idx, jnp.zeros((pad,), jnp.int32)]).reshape(CP, CHUNK)
    ef_pad = jnp.concatenate([edge_features,
                              jnp.zeros((pad, edge_features.shape[1]), F32)])
    zrows = jnp.zeros((ACC, MH), F32)

    gi = graph_idx.astype(F32)
    qp = (graph_idx ^ 1).astype(F32).reshape(N, 1)
    kg = gi.reshape(1, N)
    seg = jnp.minimum(graph_idx, n_graphs - 1).astype(F32).reshape(N, 1)

    h = _encode_nodes(node_features, W_enc_n, b_enc_n)
    for _ in range(2):
        hf, ht = _sc_gather(h, fr2d, to2d)
        att = _attention(h, qp, kg)
        m = _message_mlp(hf, ht, ef_pad, W1f, W1t, We, b1, W_m2, b_m2)
        parts = _sc_scatter(m, to2d, zrows)
        h = _gru_update(h, att, parts, War, Waz, Wan, Wxr, Wxz, Wxn,
                        Whr, Whz, Whn, br, bz, bin_, bhn)
    return _aggregate(h, seg, Wg_g, Wg_v, bg_g, bg_v, W_g2, b_g2)


# speedup vs baseline: 1.0843x; 1.0481x over previous
import functools

import jax
import jax.numpy as jnp
from jax import lax
from jax.experimental import pallas as pl
from jax.experimental.pallas import tpu as pltpu
from jax.experimental.pallas import tpu_sc as plsc

F32 = jnp.float32

N = 10000
E = 320000
NS = 32
MH = 64
GR = 128

CHUNK = 128
CP = 2560
EP = CP * CHUNK
NWORK = 32
PW = CP // NWORK
NTILE = 16
NP = 12800
ACC = 10240
NR = ACC // NTILE
GB = 8
GBG = 4


def _sc_gather(h, fr2d, to2d):
    mesh = plsc.VectorSubcoreMesh(core_axis_name="c", subcore_axis_name="s")

    @functools.partial(
        pl.kernel,
        mesh=mesh,
        out_type=(jax.ShapeDtypeStruct((EP, NS), F32),
                  jax.ShapeDtypeStruct((EP, NS), F32)),
        scratch_types=[
            pltpu.VMEM((PW, CHUNK), jnp.int32),
            pltpu.VMEM((PW, CHUNK), jnp.int32),
            pltpu.VMEM((2 * GBG * CHUNK, NS), F32),
            pltpu.VMEM((2 * GBG * CHUNK, NS), F32),
            pltpu.VMEM_SHARED((N, NS), F32),
            pltpu.SemaphoreType.DMA,
            pltpu.SemaphoreType.DMA,
        ],
        compiler_params=pltpu.CompilerParams(use_tc_tiling_on_sc=False),
    )
    def k(h_hbm, fr_hbm, to_hbm, of_hbm, ot_hbm, fidx, tidx, fbuf, tbuf,
          h_sh, s1, s2):
        c = lax.axis_index("c")
        s = lax.axis_index("s")
        hr = 640
        @pl.when(s < NTILE - 1)
        def _():
            pltpu.sync_copy(h_hbm.at[pl.ds(s * hr, hr)],
                            h_sh.at[pl.ds(s * hr, hr)])

        @pl.when(s == NTILE - 1)
        def _():
            rem = N - (NTILE - 1) * hr
            pltpu.sync_copy(h_hbm.at[pl.ds((NTILE - 1) * hr, rem)],
                            h_sh.at[pl.ds((NTILE - 1) * hr, rem)])

        plsc.subcore_barrier()
        wid = s * 2 + c
        base = wid * PW
        pltpu.sync_copy(fr_hbm.at[pl.ds(base, PW)], fidx)
        pltpu.sync_copy(to_hbm.at[pl.ds(base, PW)], tidx)

        half = GBG * CHUNK
        ng = PW // GBG

        def fire(g, off):
            for b in range(GBG):
                j = g * GBG + b
                pltpu.async_copy(h_sh.at[fidx.at[j]],
                                 fbuf.at[pl.ds(off + b * CHUNK, CHUNK)], s1)
                pltpu.async_copy(h_sh.at[tidx.at[j]],
                                 tbuf.at[pl.ds(off + b * CHUNK, CHUNK)], s2)

        def drain(off):
            for b in range(GBG):
                pltpu.make_async_copy(
                    of_hbm.at[pl.ds(0, CHUNK)],
                    fbuf.at[pl.ds(off + b * CHUNK, CHUNK)], s1).wait()
                pltpu.make_async_copy(
                    of_hbm.at[pl.ds(0, CHUNK)],
                    tbuf.at[pl.ds(off + b * CHUNK, CHUNK)], s2).wait()

        def write(g, off):
            pltpu.sync_copy(fbuf.at[pl.ds(off, half)],
                            of_hbm.at[pl.ds((base + g * GBG) * CHUNK, half)])
            pltpu.sync_copy(tbuf.at[pl.ds(off, half)],
                            ot_hbm.at[pl.ds((base + g * GBG) * CHUNK, half)])

        fire(0, 0)

        def pair(g2, carry):
            g = g2 * 2
            fire(g + 1, half)
            drain(0)
            write(g, 0)

            @pl.when(g + 2 < ng)
            def _():
                fire(g + 2, 0)

            drain(half)
            write(g + 1, half)
            return carry

        lax.fori_loop(0, ng // 2, pair, 0)

    return k(h, fr2d, to2d)


def _sc_scatter(m, to2d, zrows):
    mesh = plsc.VectorSubcoreMesh(core_axis_name="c", subcore_axis_name="s")

    @functools.partial(
        pl.kernel,
        mesh=mesh,
        out_type=jax.ShapeDtypeStruct((2 * NP, MH), F32),
        scratch_types=[
            pltpu.VMEM((PW, CHUNK), jnp.int32),
            pltpu.VMEM((2 * GBG * CHUNK, MH), F32),
            pltpu.VMEM_SHARED((ACC, MH), F32),
            pltpu.SemaphoreType.DMA,
            pltpu.SemaphoreType.DMA,
        ],
        compiler_params=pltpu.CompilerParams(use_tc_tiling_on_sc=False),
    )
    def k(m_hbm, to_hbm, z_hbm, out_hbm, tidx, mbuf, acc, sem, seml):
        c = lax.axis_index("c")
        s = lax.axis_index("s")
        pltpu.sync_copy(z_hbm.at[pl.ds(s * NR, NR)], acc.at[pl.ds(s * NR, NR)])
        plsc.subcore_barrier()
        base = (c * NTILE + s) * PW
        pltpu.sync_copy(to_hbm.at[pl.ds(base, PW)], tidx)

        half = GBG * CHUNK
        ngs = PW // GBG

        def load(g, off):
            pltpu.async_copy(m_hbm.at[pl.ds((base + g * GBG) * CHUNK, half)],
                             mbuf.at[pl.ds(off, half)], seml)

        def drain_load(off):
            pltpu.make_async_copy(m_hbm.at[pl.ds(0, half)],
                                  mbuf.at[pl.ds(off, half)], seml).wait()

        def adds(g, off):
            waits = []
            for b in range(GBG):
                waits.append(pltpu.async_copy(
                    mbuf.at[pl.ds(off + b * CHUNK, CHUNK)],
                    acc.at[tidx.at[g * GBG + b]], sem, add=True))
            for w in waits:
                w.wait()

        load(0, 0)

        def pair(g2, carry):
            g = g2 * 2
            load(g + 1, half)
            drain_load(0)
            adds(g, 0)

            @pl.when(g + 2 < ngs)
            def _():
                load(g + 2, 0)

            drain_load(half)
            adds(g + 1, half)
            return carry

        lax.fori_loop(0, ngs // 2, pair, 0)
        plsc.subcore_barrier()
        pltpu.sync_copy(acc.at[pl.ds(s * NR, NR)],
                        out_hbm.at[pl.ds(c * NP + s * NR, NR)])

    return k(m, to2d, zrows)


def _encode_nodes(nf, W, b):
    n, f = nf.shape
    blk = 2000

    def body(nf_ref, w_ref, b_ref, o_ref):
        o_ref[...] = (jnp.dot(nf_ref[...], w_ref[...],
                              preferred_element_type=F32) + b_ref[...])

    return pl.pallas_call(
        body,
        grid=(n // blk,),
        in_specs=[pl.BlockSpec((blk, f), lambda i: (i, 0)),
                  pl.BlockSpec((f, NS), lambda i: (0, 0)),
                  pl.BlockSpec((1, NS), lambda i: (0, 0))],
        out_specs=pl.BlockSpec((blk, NS), lambda i: (i, 0)),
        out_shape=jax.ShapeDtypeStruct((n, NS), F32),
    )(nf, W, b.reshape(1, NS))


def _message_mlp(hf, ht, ef, W1f, W1t, We, b1, W2, b2):
    blk = 5120
    grid = EP // blk
    fe = ef.shape[1]

    def body(hf_ref, ht_ref, ef_ref, w1f_ref, w1t_ref, we_ref, b1_ref,
             w2_ref, b2_ref, o_ref):
        z = (jnp.dot(hf_ref[...], w1f_ref[...], preferred_element_type=F32)
             + jnp.dot(ht_ref[...], w1t_ref[...], preferred_element_type=F32)
             + jnp.dot(ef_ref[...], we_ref[...], preferred_element_type=F32)
             + b1_ref[...])
        z = jnp.maximum(z, 0.0)
        m = jnp.dot(z, w2_ref[...], preferred_element_type=F32) + b2_ref[...]
        row = (lax.broadcasted_iota(jnp.int32, (blk, 1), 0)
               + pl.program_id(0) * blk)
        o_ref[...] = jnp.where(row < E, m, 0.0)

    cmap = lambda i: (0, 0)
    return pl.pallas_call(
        body,
        grid=(grid,),
        in_specs=[pl.BlockSpec((blk, NS), lambda i: (i, 0)),
                  pl.BlockSpec((blk, NS), lambda i: (i, 0)),
                  pl.BlockSpec((blk, fe), lambda i: (i, 0)),
                  pl.BlockSpec((NS, MH), cmap),
                  pl.BlockSpec((NS, MH), cmap),
                  pl.BlockSpec((fe, MH), cmap),
                  pl.BlockSpec((1, MH), cmap),
                  pl.BlockSpec((MH, MH), cmap),
                  pl.BlockSpec((1, MH), cmap)],
        out_specs=pl.BlockSpec((blk, MH), lambda i: (i, 0)),
        out_shape=jax.ShapeDtypeStruct((EP, MH), F32),
    )(hf, ht, ef, W1f, W1t, We, b1.reshape(1, MH), W2, b2.reshape(1, MH))


def _attention(h, qp, kg):
    qb = 1000
    kb = 1000
    nk = N // kb
    grid = N // qb

    def body(hq_ref, hk_ref, qp_ref, kg_ref, o_ref):
        q = hq_ref[...]
        qpv = qp_ref[...]
        qmax = jnp.max(qpv)
        qmin = jnp.min(qpv)
        n0 = jnp.sum(jnp.where(kg_ref[...] == 0.0, 1, 0))
        jlo = jnp.where(qmin == 1.0, n0 // kb, 0)
        jhi = jnp.where(qmax == 0.0, (n0 + kb - 1) // kb, nk)

        def kstep(j, carry):
            kgv = kg_ref[j]
            m_i, l_i, acc = carry
            start = pl.multiple_of(j * kb, 8)
            kblk = hk_ref[pl.ds(start, kb), :]
            s = lax.dot_general(q, kblk, (((1,), (1,)), ((), ())),
                                preferred_element_type=F32)
            s = jnp.where(qpv == kgv, s, -1e9)
            m_new = jnp.maximum(m_i, jnp.max(s, axis=1, keepdims=True))
            alpha = jnp.exp(m_i - m_new)
            p = jnp.exp(s - m_new)
            l_new = l_i * alpha + jnp.sum(p, axis=1, keepdims=True)
            acc_new = acc * alpha + jnp.dot(p, kblk,
                                            preferred_element_type=F32)
            return m_new, l_new, acc_new

        m0 = jnp.full((qb, 1), -jnp.inf, F32)
        l0 = jnp.zeros((qb, 1), F32)
        a0 = jnp.zeros((qb, NS), F32)
        _, l_f, acc = lax.fori_loop(jlo, jhi, kstep, (m0, l0, a0))
        hmean = jnp.mean(hk_ref[...], axis=0, keepdims=True)
        o_ref[...] = jnp.where(l_f > 0.0, acc / jnp.maximum(l_f, 1e-30),
                               hmean)

    cmap = lambda i: (0, 0)
    return pl.pallas_call(
        body,
        grid=(grid,),
        in_specs=[pl.BlockSpec((qb, NS), lambda i: (i, 0)),
                  pl.BlockSpec((N, NS), cmap),
                  pl.BlockSpec((qb, 1), lambda i: (i, 0)),
                  pl.BlockSpec((nk, 1, kb), lambda i: (0, 0, 0))],
        out_specs=pl.BlockSpec((qb, NS), lambda i: (i, 0)),
        out_shape=jax.ShapeDtypeStruct((N, NS), F32),
    )(h, h, qp, kg.reshape(nk, 1, kb))


def _gru_update(h, att, parts, War, Waz, Wan, Wxr, Wxz, Wxn,
                Whr, Whz, Whn, br, bz, bin_, bhn):
    qb = 400
    grid = N // qb

    def body(hq_ref, att_ref, agg0_ref, agg1_ref,
             war_ref, waz_ref, wan_ref, wxr_ref, wxz_ref, wxn_ref,
             whr_ref, whz_ref, whn_ref, br_ref, bz_ref, bin_ref, bhn_ref,
             o_ref):
        q = hq_ref[...]
        ag = agg0_ref[...] + agg1_ref[...]
        ai = q - att_ref[...]
        pre_r = (jnp.dot(ag, war_ref[...], preferred_element_type=F32)
                 + jnp.dot(ai, wxr_ref[...], preferred_element_type=F32)
                 + jnp.dot(q, whr_ref[...], preferred_element_type=F32)
                 + br_ref[...])
        pre_z = (jnp.dot(ag, waz_ref[...], preferred_element_type=F32)
                 + jnp.dot(ai, wxz_ref[...], preferred_element_type=F32)
                 + jnp.dot(q, whz_ref[...], preferred_element_type=F32)
                 + bz_ref[...])
        i_n = (jnp.dot(ag, wan_ref[...], preferred_element_type=F32)
               + jnp.dot(ai, wxn_ref[...], preferred_element_type=F32)
               + bin_ref[...])
        h_n = jnp.dot(q, whn_ref[...], preferred_element_type=F32) + bhn_ref[...]
        r = jax.nn.sigmoid(pre_r)
        zg = jax.nn.sigmoid(pre_z)
        nn = jnp.tanh(i_n + r * h_n)
        o_ref[...] = (1.0 - zg) * nn + zg * q

    cmap = lambda i: (0, 0)
    np_off = NP // qb
    return pl.pallas_call(
        body,
        grid=(grid,),
        in_specs=[pl.BlockSpec((qb, NS), lambda i: (i, 0)),
                  pl.BlockSpec((qb, NS), lambda i: (i, 0)),
                  pl.BlockSpec((qb, MH), lambda i: (i, 0)),
                  pl.BlockSpec((qb, MH), lambda i: (np_off + i, 0)),
                  pl.BlockSpec((MH, NS), cmap),
                  pl.BlockSpec((MH, NS), cmap),
                  pl.BlockSpec((MH, NS), cmap),
                  pl.BlockSpec((NS, NS), cmap),
                  pl.BlockSpec((NS, NS), cmap),
                  pl.BlockSpec((NS, NS), cmap),
                  pl.BlockSpec((NS, NS), cmap),
                  pl.BlockSpec((NS, NS), cmap),
                  pl.BlockSpec((NS, NS), cmap),
                  pl.BlockSpec((1, NS), cmap),
                  pl.BlockSpec((1, NS), cmap),
                  pl.BlockSpec((1, NS), cmap),
                  pl.BlockSpec((1, NS), cmap)],
        out_specs=pl.BlockSpec((qb, NS), lambda i: (i, 0)),
        out_shape=jax.ShapeDtypeStruct((N, NS), F32),
    )(h, att, parts, parts, War, Waz, Wan, Wxr, Wxz, Wxn, Whr, Whz, Whn,
      br.reshape(1, NS), bz.reshape(1, NS), bin_.reshape(1, NS),
      bhn.reshape(1, NS))


def _aggregate(h, seg, Wg_g, Wg_v, bg_g, bg_v, Wg2, bg2):
    blk = 2000
    grid = N // blk

    def body(h_ref, seg_ref, wgg_ref, wgv_ref, bgg_ref, bgv_ref,
             wg2_ref, bg2_ref, o_ref, acc_ref):
        i = pl.program_id(0)

        @pl.when(i == 0)
        def _():
            acc_ref[...] = jnp.zeros_like(acc_ref)

        hv = h_ref[...]
        g1 = jnp.dot(hv, wgg_ref[...], preferred_element_type=F32) + bgg_ref[...]
        g2 = jnp.dot(hv, wgv_ref[...], preferred_element_type=F32) + bgv_ref[...]
        gated = jax.nn.sigmoid(g1) * g2
        sv = seg_ref[...]
        w0 = jnp.where(sv == 0.0, 1.0, 0.0)
        w1 = jnp.where(sv == 1.0, 1.0, 0.0)
        s0 = jnp.sum(gated * w0, axis=0, keepdims=True)
        s1 = jnp.sum(gated * w1, axis=0, keepdims=True)
        acc_ref[0:1, :] = acc_ref[0:1, :] + s0
        acc_ref[1:2, :] = acc_ref[1:2, :] + s1

        @pl.when(i == grid - 1)
        def _():
            o_ref[...] = (jnp.dot(acc_ref[0:2, :], wg2_ref[...],
                                  preferred_element_type=F32) + bg2_ref[...])

    cmap = lambda i: (0, 0)
    return pl.pallas_call(
        body,
        grid=(grid,),
        in_specs=[pl.BlockSpec((blk, NS), lambda i: (i, 0)),
                  pl.BlockSpec((blk, 1), lambda i: (i, 0)),
                  pl.BlockSpec((NS, GR), cmap),
                  pl.BlockSpec((NS, GR), cmap),
                  pl.BlockSpec((1, GR), cmap),
                  pl.BlockSpec((1, GR), cmap),
                  pl.BlockSpec((GR, GR), cmap),
                  pl.BlockSpec((1, GR), cmap)],
        out_specs=pl.BlockSpec((2, GR), cmap),
        out_shape=jax.ShapeDtypeStruct((2, GR), F32),
        scratch_shapes=[pltpu.VMEM((8, GR), F32)],
    )(h, seg, Wg_g, Wg_v, bg_g.reshape(1, GR), bg_v.reshape(1, GR),
      Wg2, bg2.reshape(1, GR))


def kernel(node_features, edge_features, from_idx, to_idx, graph_idx,
           n_graphs, W_enc_n, b_enc_n, W_enc_e, b_enc_e, W_m1, b_m1,
           W_m2, b_m2, W_ih, W_hh, b_ih, b_hh, W_g1, b_g1, W_g2, b_g2):
    assert node_features.shape == (N, 128) and from_idx.shape == (E,)

    W1f = W_m1[:NS]
    W1t = W_m1[NS:2 * NS]
    W1e = W_m1[2 * NS:]
    We = W_enc_e @ W1e
    b1 = b_enc_e @ W1e + b_m1

    War, Waz, Wan = W_ih[:MH, :NS], W_ih[:MH, NS:2 * NS], W_ih[:MH, 2 * NS:]
    Wxr, Wxz, Wxn = W_ih[MH:, :NS], W_ih[MH:, NS:2 * NS], W_ih[MH:, 2 * NS:]
    Whr, Whz, Whn = W_hh[:, :NS], W_hh[:, NS:2 * NS], W_hh[:, 2 * NS:]
    br = b_ih[:NS] + b_hh[:NS]
    bz = b_ih[NS:2 * NS] + b_hh[NS:2 * NS]
    bin_ = b_ih[2 * NS:]
    bhn = b_hh[2 * NS:]

    Wg_g, Wg_v = W_g1[:, :GR], W_g1[:, GR:]
    bg_g, bg_v = b_g1[:GR], b_g1[GR:]

    pad = EP - E
    fr2d = jnp.concatenate([from_idx, jnp.zeros((pad,), jnp.int32)]).reshape(CP, CHUNK)
    to2d = jnp.concatenate([to_idx, jnp.zeros((pad,), jnp.int32)]).reshape(CP, CHUNK)
    ef_pad = jnp.concatenate([edge_features,
                              jnp.zeros((pad, edge_features.shape[1]), F32)])
    zrows = jnp.zeros((ACC, MH), F32)

    gi = graph_idx.astype(F32)
    qp = (graph_idx ^ 1).astype(F32).reshape(N, 1)
    kg = gi.reshape(1, N)
    seg = jnp.minimum(graph_idx, n_graphs - 1).astype(F32).reshape(N, 1)

    h = _encode_nodes(node_features, W_enc_n, b_enc_n)
    for _ in range(2):
        hf, ht = _sc_gather(h, fr2d, to2d)
        att = _attention(h, qp, kg)
        m = _message_mlp(hf, ht, ef_pad, W1f, W1t, We, b1, W_m2, b_m2)
        parts = _sc_scatter(m, to2d, zrows)
        h = _gru_update(h, att, parts, War, Waz, Wan, Wxr, Wxz, Wxn,
                        Whr, Whz, Whn, br, bz, bin_, bhn)
    return _aggregate(h, seg, Wg_g, Wg_v, bg_g, bg_v, W_g2, b_g2)


# speedup vs baseline: 1.0851x; 1.0007x over previous
import functools

import jax
import jax.numpy as jnp
from jax import lax
from jax.experimental import pallas as pl
from jax.experimental.pallas import tpu as pltpu
from jax.experimental.pallas import tpu_sc as plsc

F32 = jnp.float32

N = 10000
E = 320000
NS = 32
MH = 64
GR = 128

CHUNK = 128
CP = 2560
EP = CP * CHUNK
NWORK = 32
PW = CP // NWORK
NTILE = 16
NP = 12800
ACC = 10240
NR = ACC // NTILE
GB = 8
GBG = 4


def _sc_gather(h, fr2d, to2d):
    mesh = plsc.VectorSubcoreMesh(core_axis_name="c", subcore_axis_name="s")

    @functools.partial(
        pl.kernel,
        mesh=mesh,
        out_type=(jax.ShapeDtypeStruct((EP, NS), F32),
                  jax.ShapeDtypeStruct((EP, NS), F32)),
        scratch_types=[
            pltpu.VMEM((PW, CHUNK), jnp.int32),
            pltpu.VMEM((PW, CHUNK), jnp.int32),
            pltpu.VMEM((2 * GBG * CHUNK, NS), F32),
            pltpu.VMEM((2 * GBG * CHUNK, NS), F32),
            pltpu.VMEM_SHARED((N, NS), F32),
            pltpu.SemaphoreType.DMA,
            pltpu.SemaphoreType.DMA,
        ],
        compiler_params=pltpu.CompilerParams(use_tc_tiling_on_sc=False),
    )
    def k(h_hbm, fr_hbm, to_hbm, of_hbm, ot_hbm, fidx, tidx, fbuf, tbuf,
          h_sh, s1, s2):
        c = lax.axis_index("c")
        s = lax.axis_index("s")
        hr = 640
        @pl.when(s < NTILE - 1)
        def _():
            pltpu.sync_copy(h_hbm.at[pl.ds(s * hr, hr)],
                            h_sh.at[pl.ds(s * hr, hr)])

        @pl.when(s == NTILE - 1)
        def _():
            rem = N - (NTILE - 1) * hr
            pltpu.sync_copy(h_hbm.at[pl.ds((NTILE - 1) * hr, rem)],
                            h_sh.at[pl.ds((NTILE - 1) * hr, rem)])

        plsc.subcore_barrier()
        wid = s * 2 + c
        base = wid * PW
        pltpu.sync_copy(fr_hbm.at[pl.ds(base, PW)], fidx)
        pltpu.sync_copy(to_hbm.at[pl.ds(base, PW)], tidx)

        half = GBG * CHUNK
        ng = PW // GBG

        def fire(g, off):
            for b in range(GBG):
                j = g * GBG + b
                pltpu.async_copy(h_sh.at[fidx.at[j]],
                                 fbuf.at[pl.ds(off + b * CHUNK, CHUNK)], s1)
                pltpu.async_copy(h_sh.at[tidx.at[j]],
                                 tbuf.at[pl.ds(off + b * CHUNK, CHUNK)], s2)

        def drain(off):
            for b in range(GBG):
                pltpu.make_async_copy(
                    of_hbm.at[pl.ds(0, CHUNK)],
                    fbuf.at[pl.ds(off + b * CHUNK, CHUNK)], s1).wait()
                pltpu.make_async_copy(
                    of_hbm.at[pl.ds(0, CHUNK)],
                    tbuf.at[pl.ds(off + b * CHUNK, CHUNK)], s2).wait()

        def write(g, off):
            pltpu.sync_copy(fbuf.at[pl.ds(off, half)],
                            of_hbm.at[pl.ds((base + g * GBG) * CHUNK, half)])
            pltpu.sync_copy(tbuf.at[pl.ds(off, half)],
                            ot_hbm.at[pl.ds((base + g * GBG) * CHUNK, half)])

        fire(0, 0)

        def pair(g2, carry):
            g = g2 * 2
            fire(g + 1, half)
            drain(0)
            write(g, 0)

            @pl.when(g + 2 < ng)
            def _():
                fire(g + 2, 0)

            drain(half)
            write(g + 1, half)
            return carry

        lax.fori_loop(0, ng // 2, pair, 0)

    return k(h, fr2d, to2d)


def _sc_scatter(m, to2d, zrows):
    mesh = plsc.VectorSubcoreMesh(core_axis_name="c", subcore_axis_name="s")

    @functools.partial(
        pl.kernel,
        mesh=mesh,
        out_type=jax.ShapeDtypeStruct((2 * NP, MH), F32),
        scratch_types=[
            pltpu.VMEM((PW, CHUNK), jnp.int32),
            pltpu.VMEM((2 * GBG * CHUNK, MH), F32),
            pltpu.VMEM_SHARED((ACC, MH), F32),
            pltpu.SemaphoreType.DMA,
            pltpu.SemaphoreType.DMA,
        ],
        compiler_params=pltpu.CompilerParams(use_tc_tiling_on_sc=False),
    )
    def k(m_hbm, to_hbm, z_hbm, out_hbm, tidx, mbuf, acc, sem, seml):
        c = lax.axis_index("c")
        s = lax.axis_index("s")
        pltpu.sync_copy(z_hbm.at[pl.ds(s * NR, NR)], acc.at[pl.ds(s * NR, NR)])
        plsc.subcore_barrier()
        base = (c * NTILE + s) * PW
        pltpu.sync_copy(to_hbm.at[pl.ds(base, PW)], tidx)

        half = GBG * CHUNK
        ngs = PW // GBG

        def load(g, off):
            pltpu.async_copy(m_hbm.at[pl.ds((base + g * GBG) * CHUNK, half)],
                             mbuf.at[pl.ds(off, half)], seml)

        def drain_load(off):
            pltpu.make_async_copy(m_hbm.at[pl.ds(0, half)],
                                  mbuf.at[pl.ds(off, half)], seml).wait()

        def adds(g, off):
            waits = []
            for b in range(GBG):
                waits.append(pltpu.async_copy(
                    mbuf.at[pl.ds(off + b * CHUNK, CHUNK)],
                    acc.at[tidx.at[g * GBG + b]], sem, add=True))
            for w in waits:
                w.wait()

        load(0, 0)

        def pair(g2, carry):
            g = g2 * 2
            load(g + 1, half)
            drain_load(0)
            adds(g, 0)

            @pl.when(g + 2 < ngs)
            def _():
                load(g + 2, 0)

            drain_load(half)
            adds(g + 1, half)
            return carry

        lax.fori_loop(0, ngs // 2, pair, 0)
        plsc.subcore_barrier()
        pltpu.sync_copy(acc.at[pl.ds(s * NR, NR)],
                        out_hbm.at[pl.ds(c * NP + s * NR, NR)])

    return k(m, to2d, zrows)


def _encode_nodes(nf, W, b):
    n, f = nf.shape
    blk = 2000

    def body(nf_ref, w_ref, b_ref, o_ref):
        o_ref[...] = (jnp.dot(nf_ref[...], w_ref[...],
                              preferred_element_type=F32) + b_ref[...])

    return pl.pallas_call(
        body,
        grid=(n // blk,),
        in_specs=[pl.BlockSpec((blk, f), lambda i: (i, 0)),
                  pl.BlockSpec((f, NS), lambda i: (0, 0)),
                  pl.BlockSpec((1, NS), lambda i: (0, 0))],
        out_specs=pl.BlockSpec((blk, NS), lambda i: (i, 0)),
        out_shape=jax.ShapeDtypeStruct((n, NS), F32),
    )(nf, W, b.reshape(1, NS))


def _message_mlp(hf, ht, ef, W1f, W1t, We, b1, W2, b2):
    blk = 5120
    grid = EP // blk
    fe = ef.shape[1]

    def body(hf_ref, ht_ref, ef_ref, w1f_ref, w1t_ref, we_ref, b1_ref,
             w2_ref, b2_ref, o_ref):
        z = (jnp.dot(hf_ref[...], w1f_ref[...], preferred_element_type=F32)
             + jnp.dot(ht_ref[...], w1t_ref[...], preferred_element_type=F32)
             + jnp.dot(ef_ref[...], we_ref[...], preferred_element_type=F32)
             + b1_ref[...])
        z = jnp.maximum(z, 0.0)
        m = jnp.dot(z, w2_ref[...], preferred_element_type=F32) + b2_ref[...]
        row = (lax.broadcasted_iota(jnp.int32, (blk, 1), 0)
               + pl.program_id(0) * blk)
        o_ref[...] = jnp.where(row < E, m, 0.0)

    cmap = lambda i: (0, 0)
    return pl.pallas_call(
        body,
        grid=(grid,),
        in_specs=[pl.BlockSpec((blk, NS), lambda i: (i, 0)),
                  pl.BlockSpec((blk, NS), lambda i: (i, 0)),
                  pl.BlockSpec((blk, fe), lambda i: (i, 0)),
                  pl.BlockSpec((NS, MH), cmap),
                  pl.BlockSpec((NS, MH), cmap),
                  pl.BlockSpec((fe, MH), cmap),
                  pl.BlockSpec((1, MH), cmap),
                  pl.BlockSpec((MH, MH), cmap),
                  pl.BlockSpec((1, MH), cmap)],
        out_specs=pl.BlockSpec((blk, MH), lambda i: (i, 0)),
        out_shape=jax.ShapeDtypeStruct((EP, MH), F32),
    )(hf, ht, ef, W1f, W1t, We, b1.reshape(1, MH), W2, b2.reshape(1, MH))


def _attention(h, qp, kg):
    qb = 1000
    kb = 2000
    nk = N // kb
    grid = N // qb

    def body(hq_ref, hk_ref, qp_ref, kg_ref, o_ref):
        q = hq_ref[...]
        qpv = qp_ref[...]
        qmax = jnp.max(qpv)
        qmin = jnp.min(qpv)
        n0 = jnp.sum(jnp.where(kg_ref[...] == 0.0, 1, 0))
        jlo = jnp.where(qmin == 1.0, n0 // kb, 0)
        jhi = jnp.where(qmax == 0.0, (n0 + kb - 1) // kb, nk)

        def kstep(j, carry):
            kgv = kg_ref[j]
            m_i, l_i, acc = carry
            start = pl.multiple_of(j * kb, 8)
            kblk = hk_ref[pl.ds(start, kb), :]
            s = lax.dot_general(q, kblk, (((1,), (1,)), ((), ())),
                                preferred_element_type=F32)
            s = jnp.where(qpv == kgv, s, -1e9)
            m_new = jnp.maximum(m_i, jnp.max(s, axis=1, keepdims=True))
            alpha = jnp.exp(m_i - m_new)
            p = jnp.exp(s - m_new)
            l_new = l_i * alpha + jnp.sum(p, axis=1, keepdims=True)
            acc_new = acc * alpha + jnp.dot(p, kblk,
                                            preferred_element_type=F32)
            return m_new, l_new, acc_new

        m0 = jnp.full((qb, 1), -jnp.inf, F32)
        l0 = jnp.zeros((qb, 1), F32)
        a0 = jnp.zeros((qb, NS), F32)
        _, l_f, acc = lax.fori_loop(jlo, jhi, kstep, (m0, l0, a0))
        hmean = jnp.mean(hk_ref[...], axis=0, keepdims=True)
        o_ref[...] = jnp.where(l_f > 0.0, acc / jnp.maximum(l_f, 1e-30),
                               hmean)

    cmap = lambda i: (0, 0)
    return pl.pallas_call(
        body,
        grid=(grid,),
        in_specs=[pl.BlockSpec((qb, NS), lambda i: (i, 0)),
                  pl.BlockSpec((N, NS), cmap),
                  pl.BlockSpec((qb, 1), lambda i: (i, 0)),
                  pl.BlockSpec((nk, 1, kb), lambda i: (0, 0, 0))],
        out_specs=pl.BlockSpec((qb, NS), lambda i: (i, 0)),
        out_shape=jax.ShapeDtypeStruct((N, NS), F32),
    )(h, h, qp, kg.reshape(nk, 1, kb))


def _gru_update(h, att, parts, War, Waz, Wan, Wxr, Wxz, Wxn,
                Whr, Whz, Whn, br, bz, bin_, bhn):
    qb = 400
    grid = N // qb

    def body(hq_ref, att_ref, agg0_ref, agg1_ref,
             war_ref, waz_ref, wan_ref, wxr_ref, wxz_ref, wxn_ref,
             whr_ref, whz_ref, whn_ref, br_ref, bz_ref, bin_ref, bhn_ref,
             o_ref):
        q = hq_ref[...]
        ag = agg0_ref[...] + agg1_ref[...]
        ai = q - att_ref[...]
        pre_r = (jnp.dot(ag, war_ref[...], preferred_element_type=F32)
                 + jnp.dot(ai, wxr_ref[...], preferred_element_type=F32)
                 + jnp.dot(q, whr_ref[...], preferred_element_type=F32)
                 + br_ref[...])
        pre_z = (jnp.dot(ag, waz_ref[...], preferred_element_type=F32)
                 + jnp.dot(ai, wxz_ref[...], preferred_element_type=F32)
                 + jnp.dot(q, whz_ref[...], preferred_element_type=F32)
                 + bz_ref[...])
        i_n = (jnp.dot(ag, wan_ref[...], preferred_element_type=F32)
               + jnp.dot(ai, wxn_ref[...], preferred_element_type=F32)
               + bin_ref[...])
        h_n = jnp.dot(q, whn_ref[...], preferred_element_type=F32) + bhn_ref[...]
        r = jax.nn.sigmoid(pre_r)
        zg = jax.nn.sigmoid(pre_z)
        nn = jnp.tanh(i_n + r * h_n)
        o_ref[...] = (1.0 - zg) * nn + zg * q

    cmap = lambda i: (0, 0)
    np_off = NP // qb
    return pl.pallas_call(
        body,
        grid=(grid,),
        in_specs=[pl.BlockSpec((qb, NS), lambda i: (i, 0)),
                  pl.BlockSpec((qb, NS), lambda i: (i, 0)),
                  pl.BlockSpec((qb, MH), lambda i: (i, 0)),
                  pl.BlockSpec((qb, MH), lambda i: (np_off + i, 0)),
                  pl.BlockSpec((MH, NS), cmap),
                  pl.BlockSpec((MH, NS), cmap),
                  pl.BlockSpec((MH, NS), cmap),
                  pl.BlockSpec((NS, NS), cmap),
                  pl.BlockSpec((NS, NS), cmap),
                  pl.BlockSpec((NS, NS), cmap),
                  pl.BlockSpec((NS, NS), cmap),
                  pl.BlockSpec((NS, NS), cmap),
                  pl.BlockSpec((NS, NS), cmap),
                  pl.BlockSpec((1, NS), cmap),
                  pl.BlockSpec((1, NS), cmap),
                  pl.BlockSpec((1, NS), cmap),
                  pl.BlockSpec((1, NS), cmap)],
        out_specs=pl.BlockSpec((qb, NS), lambda i: (i, 0)),
        out_shape=jax.ShapeDtypeStruct((N, NS), F32),
    )(h, att, parts, parts, War, Waz, Wan, Wxr, Wxz, Wxn, Whr, Whz, Whn,
      br.reshape(1, NS), bz.reshape(1, NS), bin_.reshape(1, NS),
      bhn.reshape(1, NS))


def _aggregate(h, seg, Wg_g, Wg_v, bg_g, bg_v, Wg2, bg2):
    blk = 2000
    grid = N // blk

    def body(h_ref, seg_ref, wgg_ref, wgv_ref, bgg_ref, bgv_ref,
             wg2_ref, bg2_ref, o_ref, acc_ref):
        i = pl.program_id(0)

        @pl.when(i == 0)
        def _():
            acc_ref[...] = jnp.zeros_like(acc_ref)

        hv = h_ref[...]
        g1 = jnp.dot(hv, wgg_ref[...], preferred_element_type=F32) + bgg_ref[...]
        g2 = jnp.dot(hv, wgv_ref[...], preferred_element_type=F32) + bgv_ref[...]
        gated = jax.nn.sigmoid(g1) * g2
        sv = seg_ref[...]
        w0 = jnp.where(sv == 0.0, 1.0, 0.0)
        w1 = jnp.where(sv == 1.0, 1.0, 0.0)
        s0 = jnp.sum(gated * w0, axis=0, keepdims=True)
        s1 = jnp.sum(gated * w1, axis=0, keepdims=True)
        acc_ref[0:1, :] = acc_ref[0:1, :] + s0
        acc_ref[1:2, :] = acc_ref[1:2, :] + s1

        @pl.when(i == grid - 1)
        def _():
            o_ref[...] = (jnp.dot(acc_ref[0:2, :], wg2_ref[...],
                                  preferred_element_type=F32) + bg2_ref[...])

    cmap = lambda i: (0, 0)
    return pl.pallas_call(
        body,
        grid=(grid,),
        in_specs=[pl.BlockSpec((blk, NS), lambda i: (i, 0)),
                  pl.BlockSpec((blk, 1), lambda i: (i, 0)),
                  pl.BlockSpec((NS, GR), cmap),
                  pl.BlockSpec((NS, GR), cmap),
                  pl.BlockSpec((1, GR), cmap),
                  pl.BlockSpec((1, GR), cmap),
                  pl.BlockSpec((GR, GR), cmap),
                  pl.BlockSpec((1, GR), cmap)],
        out_specs=pl.BlockSpec((2, GR), cmap),
        out_shape=jax.ShapeDtypeStruct((2, GR), F32),
        scratch_shapes=[pltpu.VMEM((8, GR), F32)],
    )(h, seg, Wg_g, Wg_v, bg_g.reshape(1, GR), bg_v.reshape(1, GR),
      Wg2, bg2.reshape(1, GR))


def kernel(node_features, edge_features, from_idx, to_idx, graph_idx,
           n_graphs, W_enc_n, b_enc_n, W_enc_e, b_enc_e, W_m1, b_m1,
           W_m2, b_m2, W_ih, W_hh, b_ih, b_hh, W_g1, b_g1, W_g2, b_g2):
    assert node_features.shape == (N, 128) and from_idx.shape == (E,)

    W1f = W_m1[:NS]
    W1t = W_m1[NS:2 * NS]
    W1e = W_m1[2 * NS:]
    We = W_enc_e @ W1e
    b1 = b_enc_e @ W1e + b_m1

    War, Waz, Wan = W_ih[:MH, :NS], W_ih[:MH, NS:2 * NS], W_ih[:MH, 2 * NS:]
    Wxr, Wxz, Wxn = W_ih[MH:, :NS], W_ih[MH:, NS:2 * NS], W_ih[MH:, 2 * NS:]
    Whr, Whz, Whn = W_hh[:, :NS], W_hh[:, NS:2 * NS], W_hh[:, 2 * NS:]
    br = b_ih[:NS] + b_hh[:NS]
    bz = b_ih[NS:2 * NS] + b_hh[NS:2 * NS]
    bin_ = b_ih[2 * NS:]
    bhn = b_hh[2 * NS:]

    Wg_g, Wg_v = W_g1[:, :GR], W_g1[:, GR:]
    bg_g, bg_v = b_g1[:GR], b_g1[GR:]

    pad = EP - E
    fr2d = jnp.concatenate([from_idx, jnp.zeros((pad,), jnp.int32)]).reshape(CP, CHUNK)
    to2d = jnp.concatenate([to_idx, jnp.zeros((pad,), jnp.int32)]).reshape(CP, CHUNK)
    ef_pad = jnp.concatenate([edge_features,
                              jnp.zeros((pad, edge_features.shape[1]), F32)])
    zrows = jnp.zeros((ACC, MH), F32)

    gi = graph_idx.astype(F32)
    qp = (graph_idx ^ 1).astype(F32).reshape(N, 1)
    kg = gi.reshape(1, N)
    seg = jnp.minimum(graph_idx, n_graphs - 1).astype(F32).reshape(N, 1)

    h = _encode_nodes(node_features, W_enc_n, b_enc_n)
    for _ in range(2):
        hf, ht = _sc_gather(h, fr2d, to2d)
        att = _attention(h, qp, kg)
        m = _message_mlp(hf, ht, ef_pad, W1f, W1t, We, b1, W_m2, b_m2)
        parts = _sc_scatter(m, to2d, zrows)
        h = _gru_update(h, att, parts, War, Waz, Wan, Wxr, Wxz, Wxn,
                        Whr, Whz, Whn, br, bz, bin_, bhn)
    return _aggregate(h, seg, Wg_g, Wg_v, bg_g, bg_v, W_g2, b_g2)


# speedup vs baseline: 1.0880x; 1.0027x over previous
import functools

import jax
import jax.numpy as jnp
from jax import lax
from jax.experimental import pallas as pl
from jax.experimental.pallas import tpu as pltpu
from jax.experimental.pallas import tpu_sc as plsc

F32 = jnp.float32

N = 10000
E = 320000
NS = 32
MH = 64
GR = 128

CHUNK = 128
CP = 2560
EP = CP * CHUNK
NWORK = 32
PW = CP // NWORK
NTILE = 16
NP = 12800
ACC = 10240
NR = ACC // NTILE
GB = 8
GBG = 4


def _sc_gather(h, fr2d, to2d):
    mesh = plsc.VectorSubcoreMesh(core_axis_name="c", subcore_axis_name="s")

    @functools.partial(
        pl.kernel,
        mesh=mesh,
        out_type=(jax.ShapeDtypeStruct((EP, NS), F32),
                  jax.ShapeDtypeStruct((EP, NS), F32)),
        scratch_types=[
            pltpu.VMEM((PW, CHUNK), jnp.int32),
            pltpu.VMEM((PW, CHUNK), jnp.int32),
            pltpu.VMEM((2 * GBG * CHUNK, NS), F32),
            pltpu.VMEM((2 * GBG * CHUNK, NS), F32),
            pltpu.VMEM_SHARED((N, NS), F32),
            pltpu.SemaphoreType.DMA,
            pltpu.SemaphoreType.DMA,
        ],
        compiler_params=pltpu.CompilerParams(use_tc_tiling_on_sc=False),
    )
    def k(h_hbm, fr_hbm, to_hbm, of_hbm, ot_hbm, fidx, tidx, fbuf, tbuf,
          h_sh, s1, s2):
        c = lax.axis_index("c")
        s = lax.axis_index("s")
        hr = 640
        @pl.when(s < NTILE - 1)
        def _():
            pltpu.sync_copy(h_hbm.at[pl.ds(s * hr, hr)],
                            h_sh.at[pl.ds(s * hr, hr)])

        @pl.when(s == NTILE - 1)
        def _():
            rem = N - (NTILE - 1) * hr
            pltpu.sync_copy(h_hbm.at[pl.ds((NTILE - 1) * hr, rem)],
                            h_sh.at[pl.ds((NTILE - 1) * hr, rem)])

        plsc.subcore_barrier()
        wid = s * 2 + c
        base = wid * PW
        pltpu.sync_copy(fr_hbm.at[pl.ds(base, PW)], fidx)
        pltpu.sync_copy(to_hbm.at[pl.ds(base, PW)], tidx)

        half = GBG * CHUNK
        ng = PW // GBG

        def fire(g, off):
            for b in range(GBG):
                j = g * GBG + b
                pltpu.async_copy(h_sh.at[fidx.at[j]],
                                 fbuf.at[pl.ds(off + b * CHUNK, CHUNK)], s1)
                pltpu.async_copy(h_sh.at[tidx.at[j]],
                                 tbuf.at[pl.ds(off + b * CHUNK, CHUNK)], s2)

        def drain(off):
            for b in range(GBG):
                pltpu.make_async_copy(
                    of_hbm.at[pl.ds(0, CHUNK)],
                    fbuf.at[pl.ds(off + b * CHUNK, CHUNK)], s1).wait()
                pltpu.make_async_copy(
                    of_hbm.at[pl.ds(0, CHUNK)],
                    tbuf.at[pl.ds(off + b * CHUNK, CHUNK)], s2).wait()

        def write(g, off):
            pltpu.sync_copy(fbuf.at[pl.ds(off, half)],
                            of_hbm.at[pl.ds((base + g * GBG) * CHUNK, half)])
            pltpu.sync_copy(tbuf.at[pl.ds(off, half)],
                            ot_hbm.at[pl.ds((base + g * GBG) * CHUNK, half)])

        fire(0, 0)

        def pair(g2, carry):
            g = g2 * 2
            fire(g + 1, half)
            drain(0)
            write(g, 0)

            @pl.when(g + 2 < ng)
            def _():
                fire(g + 2, 0)

            drain(half)
            write(g + 1, half)
            return carry

        lax.fori_loop(0, ng // 2, pair, 0)

    return k(h, fr2d, to2d)


def _sc_scatter(m, to2d, zrows):
    mesh = plsc.VectorSubcoreMesh(core_axis_name="c", subcore_axis_name="s")

    @functools.partial(
        pl.kernel,
        mesh=mesh,
        out_type=jax.ShapeDtypeStruct((2 * NP, MH), F32),
        scratch_types=[
            pltpu.VMEM((PW, CHUNK), jnp.int32),
            pltpu.VMEM((2 * GBG * CHUNK, MH), F32),
            pltpu.VMEM_SHARED((ACC, MH), F32),
            pltpu.SemaphoreType.DMA,
            pltpu.SemaphoreType.DMA,
        ],
        compiler_params=pltpu.CompilerParams(use_tc_tiling_on_sc=False),
    )
    def k(m_hbm, to_hbm, z_hbm, out_hbm, tidx, mbuf, acc, sem, seml):
        c = lax.axis_index("c")
        s = lax.axis_index("s")
        pltpu.sync_copy(z_hbm.at[pl.ds(s * NR, NR)], acc.at[pl.ds(s * NR, NR)])
        plsc.subcore_barrier()
        base = (c * NTILE + s) * PW
        pltpu.sync_copy(to_hbm.at[pl.ds(base, PW)], tidx)

        half = GBG * CHUNK
        ngs = PW // GBG

        def load(g, off):
            pltpu.async_copy(m_hbm.at[pl.ds((base + g * GBG) * CHUNK, half)],
                             mbuf.at[pl.ds(off, half)], seml)

        def drain_load(off):
            pltpu.make_async_copy(m_hbm.at[pl.ds(0, half)],
                                  mbuf.at[pl.ds(off, half)], seml).wait()

        def adds(g, off):
            waits = []
            for b in range(GBG):
                waits.append(pltpu.async_copy(
                    mbuf.at[pl.ds(off + b * CHUNK, CHUNK)],
                    acc.at[tidx.at[g * GBG + b]], sem, add=True))
            for w in waits:
                w.wait()

        load(0, 0)

        def pair(g2, carry):
            g = g2 * 2
            load(g + 1, half)
            drain_load(0)
            adds(g, 0)

            @pl.when(g + 2 < ngs)
            def _():
                load(g + 2, 0)

            drain_load(half)
            adds(g + 1, half)
            return carry

        lax.fori_loop(0, ngs // 2, pair, 0)
        plsc.subcore_barrier()
        pltpu.sync_copy(acc.at[pl.ds(s * NR, NR)],
                        out_hbm.at[pl.ds(c * NP + s * NR, NR)])

    return k(m, to2d, zrows)


def _encode_nodes(nf, W, b):
    n, f = nf.shape
    blk = 2000

    def body(nf_ref, w_ref, b_ref, o_ref):
        o_ref[...] = (jnp.dot(nf_ref[...], w_ref[...],
                              preferred_element_type=F32) + b_ref[...])

    return pl.pallas_call(
        body,
        grid=(n // blk,),
        in_specs=[pl.BlockSpec((blk, f), lambda i: (i, 0)),
                  pl.BlockSpec((f, NS), lambda i: (0, 0)),
                  pl.BlockSpec((1, NS), lambda i: (0, 0))],
        out_specs=pl.BlockSpec((blk, NS), lambda i: (i, 0)),
        out_shape=jax.ShapeDtypeStruct((n, NS), F32),
    )(nf, W, b.reshape(1, NS))


def _message_mlp(hf, ht, ef, W1f, W1t, We, b1, W2, b2):
    blk = 10240
    grid = EP // blk
    fe = ef.shape[1]

    def body(hf_ref, ht_ref, ef_ref, w1f_ref, w1t_ref, we_ref, b1_ref,
             w2_ref, b2_ref, o_ref):
        z = (jnp.dot(hf_ref[...], w1f_ref[...], preferred_element_type=F32)
             + jnp.dot(ht_ref[...], w1t_ref[...], preferred_element_type=F32)
             + jnp.dot(ef_ref[...], we_ref[...], preferred_element_type=F32)
             + b1_ref[...])
        z = jnp.maximum(z, 0.0)
        m = jnp.dot(z, w2_ref[...], preferred_element_type=F32) + b2_ref[...]
        row = (lax.broadcasted_iota(jnp.int32, (blk, 1), 0)
               + pl.program_id(0) * blk)
        o_ref[...] = jnp.where(row < E, m, 0.0)

    cmap = lambda i: (0, 0)
    return pl.pallas_call(
        body,
        grid=(grid,),
        in_specs=[pl.BlockSpec((blk, NS), lambda i: (i, 0)),
                  pl.BlockSpec((blk, NS), lambda i: (i, 0)),
                  pl.BlockSpec((blk, fe), lambda i: (i, 0)),
                  pl.BlockSpec((NS, MH), cmap),
                  pl.BlockSpec((NS, MH), cmap),
                  pl.BlockSpec((fe, MH), cmap),
                  pl.BlockSpec((1, MH), cmap),
                  pl.BlockSpec((MH, MH), cmap),
                  pl.BlockSpec((1, MH), cmap)],
        out_specs=pl.BlockSpec((blk, MH), lambda i: (i, 0)),
        out_shape=jax.ShapeDtypeStruct((EP, MH), F32),
    )(hf, ht, ef, W1f, W1t, We, b1.reshape(1, MH), W2, b2.reshape(1, MH))


def _attention(h, qp, kg):
    qb = 1000
    kb = 2000
    nk = N // kb
    grid = N // qb

    def body(hq_ref, hk_ref, qp_ref, kg_ref, o_ref):
        q = hq_ref[...]
        qpv = qp_ref[...]
        qmax = jnp.max(qpv)
        qmin = jnp.min(qpv)
        n0 = jnp.sum(jnp.where(kg_ref[...] == 0.0, 1, 0))
        jlo = jnp.where(qmin == 1.0, n0 // kb, 0)
        jhi = jnp.where(qmax == 0.0, (n0 + kb - 1) // kb, nk)

        def kstep(j, carry):
            kgv = kg_ref[j]
            m_i, l_i, acc = carry
            start = pl.multiple_of(j * kb, 8)
            kblk = hk_ref[pl.ds(start, kb), :]
            s = lax.dot_general(q, kblk, (((1,), (1,)), ((), ())),
                                preferred_element_type=F32)
            s = jnp.where(qpv == kgv, s, -1e9)
            m_new = jnp.maximum(m_i, jnp.max(s, axis=1, keepdims=True))
            alpha = jnp.exp(m_i - m_new)
            p = jnp.exp(s - m_new)
            l_new = l_i * alpha + jnp.sum(p, axis=1, keepdims=True)
            acc_new = acc * alpha + jnp.dot(p, kblk,
                                            preferred_element_type=F32)
            return m_new, l_new, acc_new

        m0 = jnp.full((qb, 1), -jnp.inf, F32)
        l0 = jnp.zeros((qb, 1), F32)
        a0 = jnp.zeros((qb, NS), F32)
        _, l_f, acc = lax.fori_loop(jlo, jhi, kstep, (m0, l0, a0))
        hmean = jnp.mean(hk_ref[...], axis=0, keepdims=True)
        o_ref[...] = jnp.where(l_f > 0.0, acc / jnp.maximum(l_f, 1e-30),
                               hmean)

    cmap = lambda i: (0, 0)
    return pl.pallas_call(
        body,
        grid=(grid,),
        in_specs=[pl.BlockSpec((qb, NS), lambda i: (i, 0)),
                  pl.BlockSpec((N, NS), cmap),
                  pl.BlockSpec((qb, 1), lambda i: (i, 0)),
                  pl.BlockSpec((nk, 1, kb), lambda i: (0, 0, 0))],
        out_specs=pl.BlockSpec((qb, NS), lambda i: (i, 0)),
        out_shape=jax.ShapeDtypeStruct((N, NS), F32),
    )(h, h, qp, kg.reshape(nk, 1, kb))


def _gru_update(h, att, parts, War, Waz, Wan, Wxr, Wxz, Wxn,
                Whr, Whz, Whn, br, bz, bin_, bhn):
    qb = 400
    grid = N // qb

    def body(hq_ref, att_ref, agg0_ref, agg1_ref,
             war_ref, waz_ref, wan_ref, wxr_ref, wxz_ref, wxn_ref,
             whr_ref, whz_ref, whn_ref, br_ref, bz_ref, bin_ref, bhn_ref,
             o_ref):
        q = hq_ref[...]
        ag = agg0_ref[...] + agg1_ref[...]
        ai = q - att_ref[...]
        pre_r = (jnp.dot(ag, war_ref[...], preferred_element_type=F32)
                 + jnp.dot(ai, wxr_ref[...], preferred_element_type=F32)
                 + jnp.dot(q, whr_ref[...], preferred_element_type=F32)
                 + br_ref[...])
        pre_z = (jnp.dot(ag, waz_ref[...], preferred_element_type=F32)
                 + jnp.dot(ai, wxz_ref[...], preferred_element_type=F32)
                 + jnp.dot(q, whz_ref[...], preferred_element_type=F32)
                 + bz_ref[...])
        i_n = (jnp.dot(ag, wan_ref[...], preferred_element_type=F32)
               + jnp.dot(ai, wxn_ref[...], preferred_element_type=F32)
               + bin_ref[...])
        h_n = jnp.dot(q, whn_ref[...], preferred_element_type=F32) + bhn_ref[...]
        r = jax.nn.sigmoid(pre_r)
        zg = jax.nn.sigmoid(pre_z)
        nn = jnp.tanh(i_n + r * h_n)
        o_ref[...] = (1.0 - zg) * nn + zg * q

    cmap = lambda i: (0, 0)
    np_off = NP // qb
    return pl.pallas_call(
        body,
        grid=(grid,),
        in_specs=[pl.BlockSpec((qb, NS), lambda i: (i, 0)),
                  pl.BlockSpec((qb, NS), lambda i: (i, 0)),
                  pl.BlockSpec((qb, MH), lambda i: (i, 0)),
                  pl.BlockSpec((qb, MH), lambda i: (np_off + i, 0)),
                  pl.BlockSpec((MH, NS), cmap),
                  pl.BlockSpec((MH, NS), cmap),
                  pl.BlockSpec((MH, NS), cmap),
                  pl.BlockSpec((NS, NS), cmap),
                  pl.BlockSpec((NS, NS), cmap),
                  pl.BlockSpec((NS, NS), cmap),
                  pl.BlockSpec((NS, NS), cmap),
                  pl.BlockSpec((NS, NS), cmap),
                  pl.BlockSpec((NS, NS), cmap),
                  pl.BlockSpec((1, NS), cmap),
                  pl.BlockSpec((1, NS), cmap),
                  pl.BlockSpec((1, NS), cmap),
                  pl.BlockSpec((1, NS), cmap)],
        out_specs=pl.BlockSpec((qb, NS), lambda i: (i, 0)),
        out_shape=jax.ShapeDtypeStruct((N, NS), F32),
    )(h, att, parts, parts, War, Waz, Wan, Wxr, Wxz, Wxn, Whr, Whz, Whn,
      br.reshape(1, NS), bz.reshape(1, NS), bin_.reshape(1, NS),
      bhn.reshape(1, NS))


def _aggregate(h, seg, Wg_g, Wg_v, bg_g, bg_v, Wg2, bg2):
    blk = 2000
    grid = N // blk

    def body(h_ref, seg_ref, wgg_ref, wgv_ref, bgg_ref, bgv_ref,
             wg2_ref, bg2_ref, o_ref, acc_ref):
        i = pl.program_id(0)

        @pl.when(i == 0)
        def _():
            acc_ref[...] = jnp.zeros_like(acc_ref)

        hv = h_ref[...]
        g1 = jnp.dot(hv, wgg_ref[...], preferred_element_type=F32) + bgg_ref[...]
        g2 = jnp.dot(hv, wgv_ref[...], preferred_element_type=F32) + bgv_ref[...]
        gated = jax.nn.sigmoid(g1) * g2
        sv = seg_ref[...]
        w0 = jnp.where(sv == 0.0, 1.0, 0.0)
        w1 = jnp.where(sv == 1.0, 1.0, 0.0)
        s0 = jnp.sum(gated * w0, axis=0, keepdims=True)
        s1 = jnp.sum(gated * w1, axis=0, keepdims=True)
        acc_ref[0:1, :] = acc_ref[0:1, :] + s0
        acc_ref[1:2, :] = acc_ref[1:2, :] + s1

        @pl.when(i == grid - 1)
        def _():
            o_ref[...] = (jnp.dot(acc_ref[0:2, :], wg2_ref[...],
                                  preferred_element_type=F32) + bg2_ref[...])

    cmap = lambda i: (0, 0)
    return pl.pallas_call(
        body,
        grid=(grid,),
        in_specs=[pl.BlockSpec((blk, NS), lambda i: (i, 0)),
                  pl.BlockSpec((blk, 1), lambda i: (i, 0)),
                  pl.BlockSpec((NS, GR), cmap),
                  pl.BlockSpec((NS, GR), cmap),
                  pl.BlockSpec((1, GR), cmap),
                  pl.BlockSpec((1, GR), cmap),
                  pl.BlockSpec((GR, GR), cmap),
                  pl.BlockSpec((1, GR), cmap)],
        out_specs=pl.BlockSpec((2, GR), cmap),
        out_shape=jax.ShapeDtypeStruct((2, GR), F32),
        scratch_shapes=[pltpu.VMEM((8, GR), F32)],
    )(h, seg, Wg_g, Wg_v, bg_g.reshape(1, GR), bg_v.reshape(1, GR),
      Wg2, bg2.reshape(1, GR))


def kernel(node_features, edge_features, from_idx, to_idx, graph_idx,
           n_graphs, W_enc_n, b_enc_n, W_enc_e, b_enc_e, W_m1, b_m1,
           W_m2, b_m2, W_ih, W_hh, b_ih, b_hh, W_g1, b_g1, W_g2, b_g2):
    assert node_features.shape == (N, 128) and from_idx.shape == (E,)

    W1f = W_m1[:NS]
    W1t = W_m1[NS:2 * NS]
    W1e = W_m1[2 * NS:]
    We = W_enc_e @ W1e
    b1 = b_enc_e @ W1e + b_m1

    War, Waz, Wan = W_ih[:MH, :NS], W_ih[:MH, NS:2 * NS], W_ih[:MH, 2 * NS:]
    Wxr, Wxz, Wxn = W_ih[MH:, :NS], W_ih[MH:, NS:2 * NS], W_ih[MH:, 2 * NS:]
    Whr, Whz, Whn = W_hh[:, :NS], W_hh[:, NS:2 * NS], W_hh[:, 2 * NS:]
    br = b_ih[:NS] + b_hh[:NS]
    bz = b_ih[NS:2 * NS] + b_hh[NS:2 * NS]
    bin_ = b_ih[2 * NS:]
    bhn = b_hh[2 * NS:]

    Wg_g, Wg_v = W_g1[:, :GR], W_g1[:, GR:]
    bg_g, bg_v = b_g1[:GR], b_g1[GR:]

    pad = EP - E
    fr2d = jnp.concatenate([from_idx, jnp.zeros((pad,), jnp.int32)]).reshape(CP, CHUNK)
    to2d = jnp.concatenate([to_idx, jnp.zeros((pad,), jnp.int32)]).reshape(CP, CHUNK)
    ef_pad = jnp.concatenate([edge_features,
                              jnp.zeros((pad, edge_features.shape[1]), F32)])
    zrows = jnp.zeros((ACC, MH), F32)

    gi = graph_idx.astype(F32)
    qp = (graph_idx ^ 1).astype(F32).reshape(N, 1)
    kg = gi.reshape(1, N)
    seg = jnp.minimum(graph_idx, n_graphs - 1).astype(F32).reshape(N, 1)

    h = _encode_nodes(node_features, W_enc_n, b_enc_n)
    for _ in range(2):
        hf, ht = _sc_gather(h, fr2d, to2d)
        att = _attention(h, qp, kg)
        m = _message_mlp(hf, ht, ef_pad, W1f, W1t, We, b1, W_m2, b_m2)
        parts = _sc_scatter(m, to2d, zrows)
        h = _gru_update(h, att, parts, War, Waz, Wan, Wxr, Wxz, Wxn,
                        Whr, Whz, Whn, br, bz, bin_, bhn)
    return _aggregate(h, seg, Wg_g, Wg_v, bg_g, bg_v, W_g2, b_g2)
